# baseline (speedup 1.0000x reference)
import functools

import jax
import jax.numpy as jnp
from jax import lax
from jax.experimental import pallas as pl
from jax.experimental.pallas import tpu as pltpu
from jax.experimental.pallas import tpu_sc as plsc

_N_NODES = [10000, 100000, 50000, 10000]
_EM = [(0, 1, 100000), (1, 1, 100000), (1, 2, 50000), (2, 3, 20000),
       (2, 2, 50000), (1, 0, 100000), (2, 1, 50000), (3, 2, 20000)]
_HID, _HEADS, _DH, _LAYERS = 128, 8, 16, 2
_CH = 128
_ROWS_SC = 10080
_ROWS_TILE = _ROWS_SC // 16
_SWEEP = 2 * _ROWS_SC

_SRC_EDGES = {0: [0], 1: [1, 2, 5], 2: [3, 4, 6], 3: [7]}
_IN_EDGES = {0: [5], 1: [0, 1, 6], 2: [2, 4, 7], 3: [3]}


def _epad(E):
    n_chunks = -(-E // (32 * _CH))
    return 32 * _CH * n_chunks, n_chunks


def _stage1(x, wcat, bcat, K):
    N = x.shape[0]
    BN = 400

    def body(x_ref, w_ref, b_ref, *o_refs):
        xv = x_ref[...]
        for k in range(K):
            o_refs[k][...] = (
                jnp.dot(xv, w_ref[:, k * 128:(k + 1) * 128],
                        preferred_element_type=jnp.float32)
                + b_ref[0:1, k * 128:(k + 1) * 128])

    return pl.pallas_call(
        body,
        grid=(N // BN,),
        in_specs=[
            pl.BlockSpec((BN, 128), lambda i: (i, 0)),
            pl.BlockSpec((128, 128 * K), lambda i: (0, 0)),
            pl.BlockSpec((8, 128 * K), lambda i: (0, 0)),
        ],
        out_specs=[pl.BlockSpec((BN, 128), lambda i: (i, 0))] * K,
        out_shape=[jax.ShapeDtypeStruct((N, 128), jnp.float32)] * K,
    )(x, wcat, bcat)


@functools.lru_cache(maxsize=None)
def _make_pass_a(E, E_pad, n_chunks):
    def body(ke_hbm, q_hbm, src_hbm, dst_hbm, at_hbm,
             srcv, dstv, kerows, qrows, atb, sem1, sem2):
        cid = lax.axis_index("c")
        sid = lax.axis_index("s")
        wid = sid * 2 + cid
        iota16 = lax.iota(jnp.int32, 16)
        zi = jnp.zeros((16,), jnp.int32)

        def chunk_body(ci, carry):
            base = (wid * n_chunks + ci) * _CH
            pltpu.sync_copy(src_hbm.at[pl.ds(base, _CH)], srcv.at[0])
            pltpu.sync_copy(dst_hbm.at[pl.ds(base, _CH)], dstv.at[0])
            cp1 = pltpu.async_copy(ke_hbm.at[srcv.at[0]], kerows, sem1)
            cp2 = pltpu.async_copy(q_hbm.at[dstv.at[0]], qrows, sem2)
            cp1.wait()
            cp2.wait()

            def group_body(g, c2):
                rid = g * 16 + iota16
                valid = (base + rid) < E
                sv = srcv[0, pl.ds(g * 16, 16)]
                for h in range(8):
                    acc = jnp.zeros((16,), jnp.float32)
                    for j in range(16):
                        col = jnp.full((16,), h * 16 + j, jnp.int32)
                        acc = acc + (plsc.load_gather(qrows, [rid, col])
                                     * plsc.load_gather(kerows, [rid, col]))
                    ex = jnp.where(valid, jnp.exp(acc), 0.0)
                    plsc.store_scatter(atb, [rid, jnp.full((16,), h, jnp.int32)], ex)
                plsc.store_scatter(atb, [rid, jnp.full((16,), 8, jnp.int32)],
                                   plsc.bitcast(sv, jnp.float32))
                zf = jnp.zeros((16,), jnp.float32)
                for h in range(9, 16):
                    plsc.store_scatter(atb, [rid, jnp.full((16,), h, jnp.int32)], zf)
                return c2

            lax.fori_loop(0, _CH // 16, group_body, 0)
            pltpu.sync_copy(atb, at_hbm.at[pl.ds(base, _CH)])
            return carry

        lax.fori_loop(0, n_chunks, chunk_body, 0)

    return pl.kernel(
        body,
        out_type=jax.ShapeDtypeStruct((E_pad, 16), jnp.float32),
        mesh=plsc.VectorSubcoreMesh(core_axis_name="c", subcore_axis_name="s"),
        compiler_params=pltpu.CompilerParams(needs_layout_passes=False, use_tc_tiling_on_sc=False),
        scratch_types=[
            pltpu.VMEM((1, _CH), jnp.int32),
            pltpu.VMEM((1, _CH), jnp.int32),
            pltpu.VMEM((_CH, 128), jnp.float32),
            pltpu.VMEM((_CH, 128), jnp.float32),
            pltpu.VMEM((_CH, 16), jnp.float32),
            pltpu.SemaphoreType.DMA,
            pltpu.SemaphoreType.DMA,
        ],
    )


@functools.lru_cache(maxsize=None)
def _make_pass_b(E_pad, n_chunks_sc, n_sweeps):
    def body(v_hbm, dst_hbm, at_hbm, zero_hbm, agg_hbm,
             acc_sh, dstv, eid_st, ldst_st, eidb, ldstb, srcb,
             atrows, verows, wbuf, sem1, sem2):
        cid = lax.axis_index("c")
        sid = lax.axis_index("s")
        iota16 = lax.iota(jnp.int32, 16)
        zi = jnp.zeros((16,), jnp.int32)
        for t in range(9):
            eid_st[pl.ds(t * 16, 16)] = zi
            ldst_st[pl.ds(t * 16, 16)] = zi

        for s in range(n_sweeps):
            lo = (s * 2 + cid) * _ROWS_SC
            pltpu.sync_copy(zero_hbm, acc_sh.at[pl.ds(sid * _ROWS_TILE, _ROWS_TILE)])
            plsc.subcore_barrier()

            def chunk_body(ci, carry, lo=lo):
                base = (sid * n_chunks_sc + ci) * _CH
                pltpu.sync_copy(dst_hbm.at[pl.ds(base, _CH)], dstv.at[0])

                def group_body(g, off):
                    rid = g * 16 + iota16
                    dvec = dstv[0, pl.ds(g * 16, 16)]
                    loc = dvec - lo
                    m = (loc >= 0) & (loc < _ROWS_SC)
                    plsc.store_compressed(eid_st.at[pl.ds(off, 16)],
                                          base + rid, mask=m)
                    plsc.store_compressed(ldst_st.at[pl.ds(off, 16)],
                                          loc, mask=m)
                    return off + jnp.sum(m.astype(jnp.int32))

                M = lax.fori_loop(0, _CH // 16, group_body, jnp.int32(0))
                eid_st[pl.ds(M, 16)] = zi
                ldst_st[pl.ds(M, 16)] = zi
                for t in range(8):
                    sl = pl.ds(t * 16, 16)
                    eidb[0, sl] = eid_st[sl]
                    ldstb[0, sl] = ldst_st[sl]
                pltpu.async_copy(at_hbm.at[eidb.at[0]], atrows, sem1).wait()
                for t in range(8):
                    sl = t * 16 + iota16
                    sv = plsc.load_gather(atrows, [sl, jnp.full((16,), 8, jnp.int32)])
                    plsc.store_scatter(srcb, [zi, sl], plsc.bitcast(sv, jnp.int32))
                pltpu.async_copy(v_hbm.at[srcb.at[0]], verows, sem2).wait()

                def row_body(r, c2):
                    rv = jnp.full((16,), r, jnp.int32)
                    fmask = jnp.where(r < M, 1.0, 0.0)
                    av = plsc.load_gather(atrows, [rv, iota16]) * fmask
                    av = jnp.where(iota16 < 8, av, 0.0)
                    plsc.store_scatter(wbuf, [rv, 128 + iota16], av)
                    for h in range(8):
                        sc = plsc.load_gather(
                            atrows, [rv, jnp.full((16,), h, jnp.int32)]) * fmask
                        ve = plsc.load_gather(verows, [rv, h * 16 + iota16])
                        plsc.store_scatter(wbuf, [rv, h * 16 + iota16], ve * sc)
                    return c2

                lax.fori_loop(0, _CH, row_body, 0)
                pltpu.sync_copy(wbuf, acc_sh.at[ldstb.at[0]], add=True)
                return carry

            lax.fori_loop(0, n_chunks_sc, chunk_body, 0)
            plsc.subcore_barrier()
            gbase = s * _SWEEP + cid * _ROWS_SC + sid * _ROWS_TILE
            pltpu.sync_copy(acc_sh.at[pl.ds(sid * _ROWS_TILE, _ROWS_TILE)],
                            agg_hbm.at[pl.ds(gbase, _ROWS_TILE)])
            plsc.subcore_barrier()

    return pl.kernel(
        body,
        out_type=jax.ShapeDtypeStruct((n_sweeps * _SWEEP, 144), jnp.float32),
        mesh=plsc.VectorSubcoreMesh(core_axis_name="c", subcore_axis_name="s"),
        compiler_params=pltpu.CompilerParams(needs_layout_passes=False, use_tc_tiling_on_sc=False),
        scratch_types=[
            pltpu.VMEM_SHARED((_ROWS_SC, 144), jnp.float32),
            pltpu.VMEM((1, _CH), jnp.int32),
            pltpu.VMEM((_CH + 16,), jnp.int32),
            pltpu.VMEM((_CH + 16,), jnp.int32),
            pltpu.VMEM((1, _CH), jnp.int32),
            pltpu.VMEM((1, _CH), jnp.int32),
            pltpu.VMEM((1, _CH), jnp.int32),
            pltpu.VMEM((_CH, 16), jnp.float32),
            pltpu.VMEM((_CH, 128), jnp.float32),
            pltpu.VMEM((_CH, 144), jnp.float32),
            pltpu.SemaphoreType.DMA,
            pltpu.SemaphoreType.DMA,
        ],
    )


def _stage3(x, a, rmat, w3, b3, aggs, bds):
    ne = len(aggs)
    N = x.shape[0]
    BN = 400

    def body(x_ref, a_ref, r_ref, w3_ref, b3_ref, *rest):
        agg_refs = rest[:ne]
        bd_refs = rest[ne:2 * ne]
        o_ref = rest[2 * ne]
        R = r_ref[...]
        acc = jnp.zeros((BN, 128), jnp.float32)
        for i in range(ne):
            blk = agg_refs[i][...]
            num = blk[:, :128]
            den = blk[:, 128:136] + 1e-16
            denrep = jnp.dot(den, R, preferred_element_type=jnp.float32)
            acc = acc + jnp.dot(num / denrep, bd_refs[i][...],
                                preferred_element_type=jnp.float32)
        o = (jnp.dot(jax.nn.gelu(acc), w3_ref[...],
                     preferred_element_type=jnp.float32) + b3_ref[0:1, :])
        av = a_ref[0, 0]
        y = av * o + (1.0 - av) * x_ref[...]
        o_ref[...] = jnp.maximum(y, 0.0)

    in_specs = [
        pl.BlockSpec((BN, 128), lambda i: (i, 0)),
        pl.BlockSpec(memory_space=pltpu.SMEM),
        pl.BlockSpec((8, 128), lambda i: (0, 0)),
        pl.BlockSpec((128, 128), lambda i: (0, 0)),
        pl.BlockSpec((8, 128), lambda i: (0, 0)),
    ]
    in_specs += [pl.BlockSpec((BN, 144), lambda i: (i, 0)) for _ in range(ne)]
    in_specs += [pl.BlockSpec((128, 128), lambda i: (0, 0)) for _ in range(ne)]
    return pl.pallas_call(
        body,
        grid=(N // BN,),
        in_specs=in_specs,
        out_specs=pl.BlockSpec((BN, 128), lambda i: (i, 0)),
        out_shape=jax.ShapeDtypeStruct((N, 128), jnp.float32),
    )(x, a, rmat, w3, b3, *aggs, *bds)


def kernel(x_document, x_word, x_medical_concept, x_symptom_category,
           ei_contains, ei_co_occurs, ei_maps_to, ei_belongs_to, ei_related_to,
           ei_rev_contains, ei_rev_maps_to, ei_rev_belongs_to,
           lin_w, lin_b, rel_att, rel_msg, rel_pri, skip):
    xs = [x_document, x_word, x_medical_concept, x_symptom_category]
    eis = [ei_contains, ei_co_occurs, ei_maps_to, ei_belongs_to, ei_related_to,
           ei_rev_contains, ei_rev_maps_to, ei_rev_belongs_to]

    srcs, dsts, epads, nchunks = [], [], [], []
    for e, (s, d, E) in enumerate(_EM):
        ep, nc = _epad(E)
        src = eis[e][0].astype(jnp.int32)
        dst = eis[e][1].astype(jnp.int32)
        srcs.append(jnp.pad(src, (0, ep - E)))
        dsts.append(jnp.pad(dst, (0, ep - E)))
        epads.append(ep)
        nchunks.append(nc)

    eye8 = jnp.eye(8, dtype=jnp.float32)
    rmat = jnp.repeat(eye8, 16, axis=1)
    zero_init = jnp.zeros((_ROWS_TILE, 144), jnp.float32)

    def blockdiag(m):
        return jnp.einsum('hdf,hg->hdgf', m, eye8).reshape(128, 128)

    for l in range(_LAYERS):
        wke, bke, bdrm = {}, {}, {}
        for e, (s, d, E) in enumerate(_EM):
            bd_a = blockdiag(rel_att[l, e] * (rel_pri[l, e] / 4.0)[:, None, None])
            wke[e] = lin_w[l, s, 0] @ bd_a
            bke[e] = lin_b[l, s, 0] @ bd_a
            bdrm[e] = blockdiag(rel_msg[l, e])

        Q, V, KE = {}, {}, {}
        for t in range(4):
            ws = [lin_w[l, t, 1], lin_w[l, t, 2]] + [wke[e] for e in _SRC_EDGES[t]]
            bs = [lin_b[l, t, 1], lin_b[l, t, 2]] + [bke[e] for e in _SRC_EDGES[t]]
            K = len(ws)
            wcat = jnp.concatenate(ws, axis=1)
            bcat = jnp.zeros((8, 128 * K), jnp.float32).at[0].set(
                jnp.concatenate(bs, axis=0))
            outs = _stage1(xs[t], wcat, bcat, K)
            Q[t], V[t] = outs[0], outs[1]
            for j, e in enumerate(_SRC_EDGES[t]):
                KE[e] = outs[2 + j]

        AT, AGG = {}, {}
        for e, (s, d, E) in enumerate(_EM):
            AT[e] = _make_pass_a(E, epads[e], nchunks[e])(
                KE[e], Q[d], srcs[e], dsts[e])
        for e, (s, d, E) in enumerate(_EM):
            n_sweeps = -(-_N_NODES[d] // _SWEEP)
            n_chunks_sc = epads[e] // (16 * _CH)
            AGG[e] = _make_pass_b(epads[e], n_chunks_sc, n_sweeps)(
                V[s], dsts[e], AT[e], zero_init)

        new_xs = []
        for t in range(4):
            a = jax.nn.sigmoid(skip[l, t]).reshape(1, 1).astype(jnp.float32)
            b3 = jnp.zeros((8, 128), jnp.float32).at[0].set(lin_b[l, t, 3])
            aggs = [AGG[e] for e in _IN_EDGES[t]]
            bds = [bdrm[e] for e in _IN_EDGES[t]]
            new_xs.append(_stage3(xs[t], a, rmat, lin_w[l, t, 3], b3, aggs, bds))
        xs = new_xs

    return tuple(xs)


# speedup vs baseline: 13.3351x; 13.3351x over previous
import functools

import jax
import jax.numpy as jnp
from jax import lax
from jax.experimental import pallas as pl
from jax.experimental.pallas import tpu as pltpu
from jax.experimental.pallas import tpu_sc as plsc

_N_NODES = [10000, 100000, 50000, 10000]
_EM = [(0, 1, 100000), (1, 1, 100000), (1, 2, 50000), (2, 3, 20000),
       (2, 2, 50000), (1, 0, 100000), (2, 1, 50000), (3, 2, 20000)]
_HID, _HEADS, _DH, _LAYERS = 128, 8, 16, 2
_CH = 128
_ROWS_SC = 10080
_ROWS_TILE = _ROWS_SC // 16
_SWEEP = 2 * _ROWS_SC

_SRC_EDGES = {0: [0], 1: [1, 2, 5], 2: [3, 4, 6], 3: [7]}
_IN_EDGES = {0: [5], 1: [0, 1, 6], 2: [2, 4, 7], 3: [3]}


def _epad(E):
    n_chunks = -(-E // (32 * _CH))
    return 32 * _CH * n_chunks, n_chunks


def _stage1(x, wcat, bcat, widths):
    N = x.shape[0]
    BN = 400
    offs = [0]
    for w in widths:
        offs.append(offs[-1] + w)
    W = offs[-1]

    def body(x_ref, w_ref, b_ref, *o_refs):
        xv = x_ref[...]
        for k in range(len(widths)):
            o_refs[k][...] = (
                jnp.dot(xv, w_ref[:, offs[k]:offs[k + 1]],
                        preferred_element_type=jnp.float32)
                + b_ref[0:1, offs[k]:offs[k + 1]])

    return pl.pallas_call(
        body,
        grid=(N // BN,),
        in_specs=[
            pl.BlockSpec((BN, 128), lambda i: (i, 0)),
            pl.BlockSpec((128, W), lambda i: (0, 0)),
            pl.BlockSpec((8, W), lambda i: (0, 0)),
        ],
        out_specs=[pl.BlockSpec((BN, w), lambda i: (i, 0)) for w in widths],
        out_shape=[jax.ShapeDtypeStruct((N, w), jnp.float32) for w in widths],
    )(x, wcat, bcat)


@functools.lru_cache(maxsize=None)
def _make_pass_a(E, E_pad, n_chunks):
    def body(kv_hbm, q_hbm, src_hbm, dst_hbm, msg_hbm,
             srcv, dstv, kvrows, qrows, atb, msgb, sem1, sem2):
        cid = lax.axis_index("c")
        sid = lax.axis_index("s")
        wid = sid * 2 + cid
        iota16 = lax.iota(jnp.int32, 16)

        def chunk_body(ci, carry):
            base = (wid * n_chunks + ci) * _CH
            pltpu.sync_copy(src_hbm.at[pl.ds(base, _CH)], srcv.at[0])
            pltpu.sync_copy(dst_hbm.at[pl.ds(base, _CH)], dstv.at[0])
            cp1 = pltpu.async_copy(kv_hbm.at[srcv.at[0]], kvrows, sem1)
            cp2 = pltpu.async_copy(q_hbm.at[dstv.at[0]], qrows, sem2)
            cp1.wait()
            cp2.wait()

            def group_body(g, c2):
                rid = g * 16 + iota16
                valid = (base + rid) < E
                for h in range(8):
                    acc = jnp.zeros((16,), jnp.float32)
                    for j in range(16):
                        col = jnp.full((16,), h * 16 + j, jnp.int32)
                        acc = acc + (plsc.load_gather(qrows, [rid, col])
                                     * plsc.load_gather(kvrows, [rid, col]))
                    ex = jnp.where(valid, jnp.exp(acc), 0.0)
                    plsc.store_scatter(atb, [rid, jnp.full((16,), h, jnp.int32)], ex)
                return c2

            lax.fori_loop(0, _CH // 16, group_body, 0)

            def row_body(r, c2):
                rv = jnp.full((16,), r, jnp.int32)
                av = plsc.load_gather(atb, [rv, iota16])
                av = jnp.where(iota16 < 8, av, 0.0)
                plsc.store_scatter(msgb, [rv, 128 + iota16], av)
                for h in range(8):
                    sc = plsc.load_gather(atb, [rv, jnp.full((16,), h, jnp.int32)])
                    vv = plsc.load_gather(kvrows, [rv, 128 + h * 16 + iota16])
                    plsc.store_scatter(msgb, [rv, h * 16 + iota16], vv * sc)
                return c2

            lax.fori_loop(0, _CH, row_body, 0)
            pltpu.sync_copy(msgb, msg_hbm.at[pl.ds(base, _CH)])
            return carry

        lax.fori_loop(0, n_chunks, chunk_body, 0)

    return pl.kernel(
        body,
        out_type=jax.ShapeDtypeStruct((E_pad, 144), jnp.float32),
        mesh=plsc.VectorSubcoreMesh(core_axis_name="c", subcore_axis_name="s"),
        compiler_params=pltpu.CompilerParams(needs_layout_passes=False,
                                             use_tc_tiling_on_sc=False),
        scratch_types=[
            pltpu.VMEM((1, _CH), jnp.int32),
            pltpu.VMEM((1, _CH), jnp.int32),
            pltpu.VMEM((_CH, 256), jnp.float32),
            pltpu.VMEM((_CH, 128), jnp.float32),
            pltpu.VMEM((_CH, 16), jnp.float32),
            pltpu.VMEM((_CH, 144), jnp.float32),
            pltpu.SemaphoreType.DMA,
            pltpu.SemaphoreType.DMA,
        ],
    )


@functools.lru_cache(maxsize=None)
def _make_pass_b(E_pad, n_chunks_sc, n_sweeps):
    STG = 272

    def body(msg_hbm, dst_hbm, zero_hbm, agg_hbm,
             acc_sh, dstv, eid_st, ldst_st, eidb, ldstb, wbuf, sem1):
        cid = lax.axis_index("c")
        sid = lax.axis_index("s")
        iota16 = lax.iota(jnp.int32, 16)
        zi = jnp.zeros((16,), jnp.int32)
        junk = jnp.full((16,), _ROWS_SC, jnp.int32)
        for t in range(STG // 16):
            eid_st[pl.ds(t * 16, 16)] = zi
            ldst_st[pl.ds(t * 16, 16)] = junk

        def fire_batch():
            for t in range(8):
                sl = pl.ds(t * 16, 16)
                eidb[0, sl] = eid_st[sl]
                ldstb[0, sl] = ldst_st[sl]
            pltpu.async_copy(msg_hbm.at[eidb.at[0]], wbuf, sem1).wait()
            pltpu.sync_copy(wbuf, acc_sh.at[ldstb.at[0]], add=True)

        for s in range(n_sweeps):
            lo = (s * 2 + cid) * _ROWS_SC
            pltpu.sync_copy(zero_hbm, acc_sh.at[pl.ds(sid * _ROWS_TILE, _ROWS_TILE)])
            plsc.subcore_barrier()

            def chunk_body(ci, F, lo=lo):
                base = (sid * n_chunks_sc + ci) * _CH
                pltpu.sync_copy(dst_hbm.at[pl.ds(base, _CH)], dstv.at[0])

                def group_body(g, off):
                    rid = g * 16 + iota16
                    dvec = dstv[0, pl.ds(g * 16, 16)]
                    loc = dvec - lo
                    m = (loc >= 0) & (loc < _ROWS_SC)
                    plsc.store_compressed(eid_st.at[pl.ds(off, 16)],
                                          base + rid, mask=m)
                    plsc.store_compressed(ldst_st.at[pl.ds(off, 16)],
                                          loc, mask=m)
                    return off + jnp.sum(m.astype(jnp.int32))

                F = lax.fori_loop(0, _CH // 16, group_body, F)

                def with_batch(F):
                    fire_batch()
                    for t in range(8):
                        dst_sl = pl.ds(t * 16, 16)
                        src_sl = pl.ds(128 + t * 16, 16)
                        eid_st[dst_sl] = eid_st[src_sl]
                        ldst_st[dst_sl] = ldst_st[src_sl]
                    return F - 128

                return lax.cond(F >= 128, with_batch, lambda F: F, F)

            F = lax.fori_loop(0, n_chunks_sc, chunk_body, jnp.int32(0))
            for t in range(8):
                pos = t * 16 + iota16
                sl = pl.ds(t * 16, 16)
                ldst_st[sl] = jnp.where(pos < F, ldst_st[sl], junk)
            fire_batch()
            plsc.subcore_barrier()
            gbase = s * _SWEEP + cid * _ROWS_SC + sid * _ROWS_TILE
            pltpu.sync_copy(acc_sh.at[pl.ds(sid * _ROWS_TILE, _ROWS_TILE)],
                            agg_hbm.at[pl.ds(gbase, _ROWS_TILE)])
            plsc.subcore_barrier()

    return pl.kernel(
        body,
        out_type=jax.ShapeDtypeStruct((n_sweeps * _SWEEP, 144), jnp.float32),
        mesh=plsc.VectorSubcoreMesh(core_axis_name="c", subcore_axis_name="s"),
        compiler_params=pltpu.CompilerParams(needs_layout_passes=False,
                                             use_tc_tiling_on_sc=False),
        scratch_types=[
            pltpu.VMEM_SHARED((_ROWS_SC + 8, 144), jnp.float32),
            pltpu.VMEM((1, _CH), jnp.int32),
            pltpu.VMEM((STG,), jnp.int32),
            pltpu.VMEM((STG,), jnp.int32),
            pltpu.VMEM((1, _CH), jnp.int32),
            pltpu.VMEM((1, _CH), jnp.int32),
            pltpu.VMEM((_CH, 144), jnp.float32),
            pltpu.SemaphoreType.DMA,
        ],
    )


def _stage3(x, a, rmat, w3, b3, aggs, bds):
    ne = len(aggs)
    N = x.shape[0]
    BN = 400

    def body(x_ref, a_ref, r_ref, w3_ref, b3_ref, *rest):
        agg_refs = rest[:ne]
        bd_refs = rest[ne:2 * ne]
        o_ref = rest[2 * ne]
        R = r_ref[...]
        acc = jnp.zeros((BN, 128), jnp.float32)
        for i in range(ne):
            blk = agg_refs[i][...]
            num = blk[:, :128]
            den = blk[:, 128:136] + 1e-16
            denrep = jnp.dot(den, R, preferred_element_type=jnp.float32)
            acc = acc + jnp.dot(num / denrep, bd_refs[i][...],
                                preferred_element_type=jnp.float32)
        o = (jnp.dot(jax.nn.gelu(acc), w3_ref[...],
                     preferred_element_type=jnp.float32) + b3_ref[0:1, :])
        av = a_ref[0, 0]
        y = av * o + (1.0 - av) * x_ref[...]
        o_ref[...] = jnp.maximum(y, 0.0)

    in_specs = [
        pl.BlockSpec((BN, 128), lambda i: (i, 0)),
        pl.BlockSpec(memory_space=pltpu.SMEM),
        pl.BlockSpec((8, 128), lambda i: (0, 0)),
        pl.BlockSpec((128, 128), lambda i: (0, 0)),
        pl.BlockSpec((8, 128), lambda i: (0, 0)),
    ]
    in_specs += [pl.BlockSpec((BN, 144), lambda i: (i, 0)) for _ in range(ne)]
    in_specs += [pl.BlockSpec((128, 128), lambda i: (0, 0)) for _ in range(ne)]
    return pl.pallas_call(
        body,
        grid=(N // BN,),
        in_specs=in_specs,
        out_specs=pl.BlockSpec((BN, 128), lambda i: (i, 0)),
        out_shape=jax.ShapeDtypeStruct((N, 128), jnp.float32),
    )(x, a, rmat, w3, b3, *aggs, *bds)


def kernel(x_document, x_word, x_medical_concept, x_symptom_category,
           ei_contains, ei_co_occurs, ei_maps_to, ei_belongs_to, ei_related_to,
           ei_rev_contains, ei_rev_maps_to, ei_rev_belongs_to,
           lin_w, lin_b, rel_att, rel_msg, rel_pri, skip):
    xs = [x_document, x_word, x_medical_concept, x_symptom_category]
    eis = [ei_contains, ei_co_occurs, ei_maps_to, ei_belongs_to, ei_related_to,
           ei_rev_contains, ei_rev_maps_to, ei_rev_belongs_to]

    srcs, dsts, epads, nchunks = [], [], [], []
    for e, (s, d, E) in enumerate(_EM):
        ep, nc = _epad(E)
        src = eis[e][0].astype(jnp.int32)
        dst = eis[e][1].astype(jnp.int32)
        srcs.append(jnp.pad(src, (0, ep - E)))
        dsts.append(jnp.pad(dst, (0, ep - E)))
        epads.append(ep)
        nchunks.append(nc)

    eye8 = jnp.eye(8, dtype=jnp.float32)
    rmat = jnp.repeat(eye8, 16, axis=1)
    zero_init = jnp.zeros((_ROWS_TILE, 144), jnp.float32)

    def blockdiag(m):
        return jnp.einsum('hdf,hg->hdgf', m, eye8).reshape(128, 128)

    for l in range(_LAYERS):
        wke, bke, bdrm = {}, {}, {}
        for e, (s, d, E) in enumerate(_EM):
            bd_a = blockdiag(rel_att[l, e] * (rel_pri[l, e] / 4.0)[:, None, None])
            wke[e] = lin_w[l, s, 0] @ bd_a
            bke[e] = lin_b[l, s, 0] @ bd_a
            bdrm[e] = blockdiag(rel_msg[l, e])

        Q, KV = {}, {}
        for t in range(4):
            ws = [lin_w[l, t, 1]]
            bs = [lin_b[l, t, 1]]
            widths = [128]
            for e in _SRC_EDGES[t]:
                ws += [wke[e], lin_w[l, t, 2]]
                bs += [bke[e], lin_b[l, t, 2]]
                widths.append(256)
            wcat = jnp.concatenate(ws, axis=1)
            bcat = jnp.zeros((8, wcat.shape[1]), jnp.float32).at[0].set(
                jnp.concatenate(bs, axis=0))
            outs = _stage1(xs[t], wcat, bcat, tuple(widths))
            Q[t] = outs[0]
            for j, e in enumerate(_SRC_EDGES[t]):
                KV[e] = outs[1 + j]

        MSG, AGG = {}, {}
        for e, (s, d, E) in enumerate(_EM):
            MSG[e] = _make_pass_a(E, epads[e], nchunks[e])(
                KV[e], Q[d], srcs[e], dsts[e])
        for e, (s, d, E) in enumerate(_EM):
            n_sweeps = -(-_N_NODES[d] // _SWEEP)
            n_chunks_sc = epads[e] // (16 * _CH)
            AGG[e] = _make_pass_b(epads[e], n_chunks_sc, n_sweeps)(
                MSG[e], dsts[e], zero_init)

        new_xs = []
        for t in range(4):
            a = jax.nn.sigmoid(skip[l, t]).reshape(1, 1).astype(jnp.float32)
            b3 = jnp.zeros((8, 128), jnp.float32).at[0].set(lin_b[l, t, 3])
            aggs = [AGG[e] for e in _IN_EDGES[t]]
            bds = [bdrm[e] for e in _IN_EDGES[t]]
            new_xs.append(_stage3(xs[t], a, rmat, lin_w[l, t, 3], b3, aggs, bds))
        xs = new_xs

    return tuple(xs)


# speedup vs baseline: 15.0134x; 1.1259x over previous
import functools

import jax
import jax.numpy as jnp
from jax import lax
from jax.experimental import pallas as pl
from jax.experimental.pallas import tpu as pltpu
from jax.experimental.pallas import tpu_sc as plsc

_N_NODES = [10000, 100000, 50000, 10000]
_EM = [(0, 1, 100000), (1, 1, 100000), (1, 2, 50000), (2, 3, 20000),
       (2, 2, 50000), (1, 0, 100000), (2, 1, 50000), (3, 2, 20000)]
_HID, _HEADS, _DH, _LAYERS = 128, 8, 16, 2
_CH = 128
_ROWS_SC = 10080
_ROWS_TILE = _ROWS_SC // 16
_SWEEP = 2 * _ROWS_SC

_SRC_EDGES = {0: [0], 1: [1, 2, 5], 2: [3, 4, 6], 3: [7]}
_IN_EDGES = {0: [5], 1: [0, 1, 6], 2: [2, 4, 7], 3: [3]}


def _epad(E):
    n_chunks = -(-E // (32 * _CH))
    return 32 * _CH * n_chunks, n_chunks


def _stage1(x, wcat, bcat, widths):
    N = x.shape[0]
    BN = 400
    offs = [0]
    for w in widths:
        offs.append(offs[-1] + w)
    W = offs[-1]

    def body(x_ref, w_ref, b_ref, *o_refs):
        xv = x_ref[...]
        for k in range(len(widths)):
            o_refs[k][...] = (
                jnp.dot(xv, w_ref[:, offs[k]:offs[k + 1]],
                        preferred_element_type=jnp.float32)
                + b_ref[0:1, offs[k]:offs[k + 1]])

    return pl.pallas_call(
        body,
        grid=(N // BN,),
        in_specs=[
            pl.BlockSpec((BN, 128), lambda i: (i, 0)),
            pl.BlockSpec((128, W), lambda i: (0, 0)),
            pl.BlockSpec((8, W), lambda i: (0, 0)),
        ],
        out_specs=[pl.BlockSpec((BN, w), lambda i: (i, 0)) for w in widths],
        out_shape=[jax.ShapeDtypeStruct((N, w), jnp.float32) for w in widths],
    )(x, wcat, bcat)


@functools.lru_cache(maxsize=None)
def _make_pass_a(E, E_pad, n_chunks):
    def body(kv_hbm, q_hbm, src_hbm, dst_hbm, msg_hbm,
             srcv0, dstv0, kvrows0, qrows0, srcv1, dstv1, kvrows1, qrows1,
             atb, msgb, *sems):
        cid = lax.axis_index("c")
        sid = lax.axis_index("s")
        wid = sid * 2 + cid
        iota16 = lax.iota(jnp.int32, 16)
        bufs = [(srcv0, dstv0, kvrows0, qrows0, sems[0:4]),
                (srcv1, dstv1, kvrows1, qrows1, sems[4:8])]

        def cbase(ci):
            return (wid * n_chunks + ci) * _CH

        def issue_idx(ci, p):
            srcv, dstv, _, _, (si, di, _, _) = bufs[p]
            pltpu.async_copy(src_hbm.at[pl.ds(cbase(ci), _CH)], srcv.at[0], si)
            pltpu.async_copy(dst_hbm.at[pl.ds(cbase(ci), _CH)], dstv.at[0], di)

        def issue_gather(ci, p):
            srcv, dstv, kvrows, qrows, (si, di, gk, gq) = bufs[p]
            pltpu.make_async_copy(src_hbm.at[pl.ds(cbase(ci), _CH)],
                                  srcv.at[0], si).wait()
            pltpu.make_async_copy(dst_hbm.at[pl.ds(cbase(ci), _CH)],
                                  dstv.at[0], di).wait()
            pltpu.async_copy(kv_hbm.at[srcv.at[0]], kvrows, gk)
            pltpu.async_copy(q_hbm.at[dstv.at[0]], qrows, gq)

        def compute(ci, p):
            srcv, dstv, kvrows, qrows, (si, di, gk, gq) = bufs[p]
            base = cbase(ci)
            pltpu.make_async_copy(kv_hbm.at[srcv.at[0]], kvrows, gk).wait()
            pltpu.make_async_copy(q_hbm.at[dstv.at[0]], qrows, gq).wait()

            def group_body(g, c2):
                rid = g * 16 + iota16
                valid = (base + rid) < E
                for h in range(8):
                    acc = jnp.zeros((16,), jnp.float32)
                    for j in range(16):
                        col = jnp.full((16,), h * 16 + j, jnp.int32)
                        acc = acc + (plsc.load_gather(qrows, [rid, col])
                                     * plsc.load_gather(kvrows, [rid, col]))
                    ex = jnp.where(valid, jnp.exp(acc), 0.0)
                    plsc.store_scatter(atb, [rid, jnp.full((16,), h, jnp.int32)], ex)
                return c2

            lax.fori_loop(0, _CH // 16, group_body, 0)

            def row_body(r, c2):
                rv = jnp.full((16,), r, jnp.int32)
                av = plsc.load_gather(atb, [rv, iota16])
                av = jnp.where(iota16 < 8, av, 0.0)
                plsc.store_scatter(msgb, [rv, 128 + iota16], av)
                for h in range(8):
                    sc = plsc.load_gather(atb, [rv, jnp.full((16,), h, jnp.int32)])
                    vv = plsc.load_gather(kvrows, [rv, 128 + h * 16 + iota16])
                    plsc.store_scatter(msgb, [rv, h * 16 + iota16], vv * sc)
                return c2

            lax.fori_loop(0, _CH, row_body, 0)
            pltpu.sync_copy(msgb, msg_hbm.at[pl.ds(base, _CH)])

        issue_idx(0, 0)
        issue_gather(0, 0)

        def step(k, carry):
            c0 = 2 * k
            c1 = 2 * k + 1

            @pl.when(c1 < n_chunks)
            def _():
                issue_idx(c1, 1)
                issue_gather(c1, 1)
            compute(c0, 0)

            @pl.when(c1 < n_chunks)
            def _():
                @pl.when(c1 + 1 < n_chunks)
                def _():
                    issue_idx(c1 + 1, 0)
                    issue_gather(c1 + 1, 0)
                compute(c1, 1)
            return carry

        lax.fori_loop(0, (n_chunks + 1) // 2, step, 0)

    return pl.kernel(
        body,
        out_type=jax.ShapeDtypeStruct((E_pad, 144), jnp.float32),
        mesh=plsc.VectorSubcoreMesh(core_axis_name="c", subcore_axis_name="s"),
        compiler_params=pltpu.CompilerParams(needs_layout_passes=False,
                                             use_tc_tiling_on_sc=False),
        scratch_types=[
            pltpu.VMEM((1, _CH), jnp.int32),
            pltpu.VMEM((1, _CH), jnp.int32),
            pltpu.VMEM((_CH, 256), jnp.float32),
            pltpu.VMEM((_CH, 128), jnp.float32),
            pltpu.VMEM((1, _CH), jnp.int32),
            pltpu.VMEM((1, _CH), jnp.int32),
            pltpu.VMEM((_CH, 256), jnp.float32),
            pltpu.VMEM((_CH, 128), jnp.float32),
            pltpu.VMEM((_CH, 16), jnp.float32),
            pltpu.VMEM((_CH, 144), jnp.float32),
        ] + [pltpu.SemaphoreType.DMA] * 8,
    )


@functools.lru_cache(maxsize=None)
def _make_pass_b(E_pad, n_chunks_sc, n_sweeps):
    STG = 272

    def body(msg_hbm, dst_hbm, zero_hbm, agg_hbm,
             acc_sh, dstv, eid_st, ldst_st, eidb, ldstb, wbuf, sem1):
        cid = lax.axis_index("c")
        sid = lax.axis_index("s")
        iota16 = lax.iota(jnp.int32, 16)
        zi = jnp.zeros((16,), jnp.int32)
        junk = jnp.full((16,), _ROWS_SC, jnp.int32)
        for t in range(STG // 16):
            eid_st[pl.ds(t * 16, 16)] = zi
            ldst_st[pl.ds(t * 16, 16)] = junk

        def fire_batch():
            for t in range(8):
                sl = pl.ds(t * 16, 16)
                eidb[0, sl] = eid_st[sl]
                ldstb[0, sl] = ldst_st[sl]
            pltpu.async_copy(msg_hbm.at[eidb.at[0]], wbuf, sem1).wait()
            pltpu.sync_copy(wbuf, acc_sh.at[ldstb.at[0]], add=True)

        for s in range(n_sweeps):
            lo = (s * 2 + cid) * _ROWS_SC
            pltpu.sync_copy(zero_hbm, acc_sh.at[pl.ds(sid * _ROWS_TILE, _ROWS_TILE)])
            plsc.subcore_barrier()

            def chunk_body(ci, F, lo=lo):
                base = (sid * n_chunks_sc + ci) * _CH
                pltpu.sync_copy(dst_hbm.at[pl.ds(base, _CH)], dstv.at[0])

                def group_body(g, off):
                    rid = g * 16 + iota16
                    dvec = dstv[0, pl.ds(g * 16, 16)]
                    loc = dvec - lo
                    m = (loc >= 0) & (loc < _ROWS_SC)
                    plsc.store_compressed(eid_st.at[pl.ds(off, 16)],
                                          base + rid, mask=m)
                    plsc.store_compressed(ldst_st.at[pl.ds(off, 16)],
                                          loc, mask=m)
                    return off + jnp.sum(m.astype(jnp.int32))

                F = lax.fori_loop(0, _CH // 16, group_body, F)

                def with_batch(F):
                    fire_batch()
                    for t in range(8):
                        dst_sl = pl.ds(t * 16, 16)
                        src_sl = pl.ds(128 + t * 16, 16)
                        eid_st[dst_sl] = eid_st[src_sl]
                        ldst_st[dst_sl] = ldst_st[src_sl]
                    return F - 128

                return lax.cond(F >= 128, with_batch, lambda F: F, F)

            F = lax.fori_loop(0, n_chunks_sc, chunk_body, jnp.int32(0))
            for t in range(8):
                pos = t * 16 + iota16
                sl = pl.ds(t * 16, 16)
                ldst_st[sl] = jnp.where(pos < F, ldst_st[sl], junk)
            fire_batch()
            plsc.subcore_barrier()
            gbase = s * _SWEEP + cid * _ROWS_SC + sid * _ROWS_TILE
            pltpu.sync_copy(acc_sh.at[pl.ds(sid * _ROWS_TILE, _ROWS_TILE)],
                            agg_hbm.at[pl.ds(gbase, _ROWS_TILE)])
            plsc.subcore_barrier()

    return pl.kernel(
        body,
        out_type=jax.ShapeDtypeStruct((n_sweeps * _SWEEP, 144), jnp.float32),
        mesh=plsc.VectorSubcoreMesh(core_axis_name="c", subcore_axis_name="s"),
        compiler_params=pltpu.CompilerParams(needs_layout_passes=False,
                                             use_tc_tiling_on_sc=False),
        scratch_types=[
            pltpu.VMEM_SHARED((_ROWS_SC + 8, 144), jnp.float32),
            pltpu.VMEM((1, _CH), jnp.int32),
            pltpu.VMEM((STG,), jnp.int32),
            pltpu.VMEM((STG,), jnp.int32),
            pltpu.VMEM((1, _CH), jnp.int32),
            pltpu.VMEM((1, _CH), jnp.int32),
            pltpu.VMEM((_CH, 144), jnp.float32),
            pltpu.SemaphoreType.DMA,
        ],
    )


def _stage3(x, a, rmat, w3, b3, aggs, bds):
    ne = len(aggs)
    N = x.shape[0]
    BN = 400

    def body(x_ref, a_ref, r_ref, w3_ref, b3_ref, *rest):
        agg_refs = rest[:ne]
        bd_refs = rest[ne:2 * ne]
        o_ref = rest[2 * ne]
        R = r_ref[...]
        acc = jnp.zeros((BN, 128), jnp.float32)
        for i in range(ne):
            blk = agg_refs[i][...]
            num = blk[:, :128]
            den = blk[:, 128:136] + 1e-16
            denrep = jnp.dot(den, R, preferred_element_type=jnp.float32)
            acc = acc + jnp.dot(num / denrep, bd_refs[i][...],
                                preferred_element_type=jnp.float32)
        o = (jnp.dot(jax.nn.gelu(acc), w3_ref[...],
                     preferred_element_type=jnp.float32) + b3_ref[0:1, :])
        av = a_ref[0, 0]
        y = av * o + (1.0 - av) * x_ref[...]
        o_ref[...] = jnp.maximum(y, 0.0)

    in_specs = [
        pl.BlockSpec((BN, 128), lambda i: (i, 0)),
        pl.BlockSpec(memory_space=pltpu.SMEM),
        pl.BlockSpec((8, 128), lambda i: (0, 0)),
        pl.BlockSpec((128, 128), lambda i: (0, 0)),
        pl.BlockSpec((8, 128), lambda i: (0, 0)),
    ]
    in_specs += [pl.BlockSpec((BN, 144), lambda i: (i, 0)) for _ in range(ne)]
    in_specs += [pl.BlockSpec((128, 128), lambda i: (0, 0)) for _ in range(ne)]
    return pl.pallas_call(
        body,
        grid=(N // BN,),
        in_specs=in_specs,
        out_specs=pl.BlockSpec((BN, 128), lambda i: (i, 0)),
        out_shape=jax.ShapeDtypeStruct((N, 128), jnp.float32),
    )(x, a, rmat, w3, b3, *aggs, *bds)


def kernel(x_document, x_word, x_medical_concept, x_symptom_category,
           ei_contains, ei_co_occurs, ei_maps_to, ei_belongs_to, ei_related_to,
           ei_rev_contains, ei_rev_maps_to, ei_rev_belongs_to,
           lin_w, lin_b, rel_att, rel_msg, rel_pri, skip):
    xs = [x_document, x_word, x_medical_concept, x_symptom_category]
    eis = [ei_contains, ei_co_occurs, ei_maps_to, ei_belongs_to, ei_related_to,
           ei_rev_contains, ei_rev_maps_to, ei_rev_belongs_to]

    srcs, dsts, epads, nchunks = [], [], [], []
    for e, (s, d, E) in enumerate(_EM):
        ep, nc = _epad(E)
        src = eis[e][0].astype(jnp.int32)
        dst = eis[e][1].astype(jnp.int32)
        srcs.append(jnp.pad(src, (0, ep - E)))
        dsts.append(jnp.pad(dst, (0, ep - E)))
        epads.append(ep)
        nchunks.append(nc)

    eye8 = jnp.eye(8, dtype=jnp.float32)
    rmat = jnp.repeat(eye8, 16, axis=1)
    zero_init = jnp.zeros((_ROWS_TILE, 144), jnp.float32)

    def blockdiag(m):
        return jnp.einsum('hdf,hg->hdgf', m, eye8).reshape(128, 128)

    for l in range(_LAYERS):
        wke, bke, bdrm = {}, {}, {}
        for e, (s, d, E) in enumerate(_EM):
            bd_a = blockdiag(rel_att[l, e] * (rel_pri[l, e] / 4.0)[:, None, None])
            wke[e] = lin_w[l, s, 0] @ bd_a
            bke[e] = lin_b[l, s, 0] @ bd_a
            bdrm[e] = blockdiag(rel_msg[l, e])

        Q, KV = {}, {}
        for t in range(4):
            ws = [lin_w[l, t, 1]]
            bs = [lin_b[l, t, 1]]
            widths = [128]
            for e in _SRC_EDGES[t]:
                ws += [wke[e], lin_w[l, t, 2]]
                bs += [bke[e], lin_b[l, t, 2]]
                widths.append(256)
            wcat = jnp.concatenate(ws, axis=1)
            bcat = jnp.zeros((8, wcat.shape[1]), jnp.float32).at[0].set(
                jnp.concatenate(bs, axis=0))
            outs = _stage1(xs[t], wcat, bcat, tuple(widths))
            Q[t] = outs[0]
            for j, e in enumerate(_SRC_EDGES[t]):
                KV[e] = outs[1 + j]

        MSG, AGG = {}, {}
        for e, (s, d, E) in enumerate(_EM):
            MSG[e] = _make_pass_a(E, epads[e], nchunks[e])(
                KV[e], Q[d], srcs[e], dsts[e])
        for e, (s, d, E) in enumerate(_EM):
            n_sweeps = -(-_N_NODES[d] // _SWEEP)
            n_chunks_sc = epads[e] // (16 * _CH)
            AGG[e] = _make_pass_b(epads[e], n_chunks_sc, n_sweeps)(
                MSG[e], dsts[e], zero_init)

        new_xs = []
        for t in range(4):
            a = jax.nn.sigmoid(skip[l, t]).reshape(1, 1).astype(jnp.float32)
            b3 = jnp.zeros((8, 128), jnp.float32).at[0].set(lin_b[l, t, 3])
            aggs = [AGG[e] for e in _IN_EDGES[t]]
            bds = [bdrm[e] for e in _IN_EDGES[t]]
            new_xs.append(_stage3(xs[t], a, rmat, lin_w[l, t, 3], b3, aggs, bds))
        xs = new_xs

    return tuple(xs)


# speedup vs baseline: 15.1888x; 1.0117x over previous
import functools

import jax
import jax.numpy as jnp
from jax import lax
from jax.experimental import pallas as pl
from jax.experimental.pallas import tpu as pltpu
from jax.experimental.pallas import tpu_sc as plsc

_N_NODES = [10000, 100000, 50000, 10000]
_EM = [(0, 1, 100000), (1, 1, 100000), (1, 2, 50000), (2, 3, 20000),
       (2, 2, 50000), (1, 0, 100000), (2, 1, 50000), (3, 2, 20000)]
_HID, _HEADS, _DH, _LAYERS = 128, 8, 16, 2
_CH = 128
_ROWS_SC = 10080
_ROWS_TILE = _ROWS_SC // 16
_SWEEP = 2 * _ROWS_SC

_SRC_EDGES = {0: [0], 1: [1, 2, 5], 2: [3, 4, 6], 3: [7]}
_IN_EDGES = {0: [5], 1: [0, 1, 6], 2: [2, 4, 7], 3: [3]}


def _epad(E):
    n_chunks = -(-E // (32 * _CH))
    return 32 * _CH * n_chunks, n_chunks


def _stage1(x, wcat, bcat, widths):
    N = x.shape[0]
    BN = 400
    offs = [0]
    for w in widths:
        offs.append(offs[-1] + w)
    W = offs[-1]

    def body(x_ref, w_ref, b_ref, *o_refs):
        xv = x_ref[...]
        for k in range(len(widths)):
            o_refs[k][...] = (
                jnp.dot(xv, w_ref[:, offs[k]:offs[k + 1]],
                        preferred_element_type=jnp.float32)
                + b_ref[0:1, offs[k]:offs[k + 1]])

    return pl.pallas_call(
        body,
        grid=(N // BN,),
        in_specs=[
            pl.BlockSpec((BN, 128), lambda i: (i, 0)),
            pl.BlockSpec((128, W), lambda i: (0, 0)),
            pl.BlockSpec((8, W), lambda i: (0, 0)),
        ],
        out_specs=[pl.BlockSpec((BN, w), lambda i: (i, 0)) for w in widths],
        out_shape=[jax.ShapeDtypeStruct((N, w), jnp.float32) for w in widths],
    )(x, wcat, bcat)


@functools.lru_cache(maxsize=None)
def _make_pass_a(E, E_pad, n_chunks):
    def body(kv_hbm, q_hbm, src_hbm, dst_hbm, msg_hbm,
             srcv0, dstv0, kvrows0, qrows0, srcv1, dstv1, kvrows1, qrows1,
             atb, msgb, *sems):
        cid = lax.axis_index("c")
        sid = lax.axis_index("s")
        wid = sid * 2 + cid
        iota16 = lax.iota(jnp.int32, 16)
        bufs = [(srcv0, dstv0, kvrows0, qrows0, sems[0:4]),
                (srcv1, dstv1, kvrows1, qrows1, sems[4:8])]

        def cbase(ci):
            return (wid * n_chunks + ci) * _CH

        def issue_idx(ci, p):
            srcv, dstv, _, _, (si, di, _, _) = bufs[p]
            pltpu.async_copy(src_hbm.at[pl.ds(cbase(ci), _CH)], srcv.at[0], si)
            pltpu.async_copy(dst_hbm.at[pl.ds(cbase(ci), _CH)], dstv.at[0], di)

        def issue_gather(ci, p):
            srcv, dstv, kvrows, qrows, (si, di, gk, gq) = bufs[p]
            pltpu.make_async_copy(src_hbm.at[pl.ds(cbase(ci), _CH)],
                                  srcv.at[0], si).wait()
            pltpu.make_async_copy(dst_hbm.at[pl.ds(cbase(ci), _CH)],
                                  dstv.at[0], di).wait()
            pltpu.async_copy(kv_hbm.at[srcv.at[0]], kvrows, gk)
            pltpu.async_copy(q_hbm.at[dstv.at[0]], qrows, gq)

        def compute(ci, p):
            srcv, dstv, kvrows, qrows, (si, di, gk, gq) = bufs[p]
            base = cbase(ci)
            pltpu.make_async_copy(kv_hbm.at[srcv.at[0]], kvrows, gk).wait()
            pltpu.make_async_copy(q_hbm.at[dstv.at[0]], qrows, gq).wait()

            def group_body(g, c2):
                rid = g * 16 + iota16
                valid = (base + rid) < E
                for h in range(8):
                    acc = jnp.zeros((16,), jnp.float32)
                    for j in range(16):
                        col = jnp.full((16,), h * 16 + j, jnp.int32)
                        acc = acc + (plsc.load_gather(qrows, [rid, col])
                                     * plsc.load_gather(kvrows, [rid, col]))
                    ex = jnp.where(valid, jnp.exp(acc), 0.0)
                    plsc.store_scatter(atb, [rid, jnp.full((16,), h, jnp.int32)], ex)
                return c2

            lax.fori_loop(0, _CH // 16, group_body, 0)

            def row_body(r, c2):
                rv = jnp.full((16,), r, jnp.int32)
                av = plsc.load_gather(atb, [rv, iota16])
                av = jnp.where(iota16 < 8, av, 0.0)
                plsc.store_scatter(msgb, [rv, 128 + iota16], av)
                for h in range(8):
                    sc = plsc.load_gather(atb, [rv, jnp.full((16,), h, jnp.int32)])
                    vv = plsc.load_gather(kvrows, [rv, 128 + h * 16 + iota16])
                    plsc.store_scatter(msgb, [rv, h * 16 + iota16], vv * sc)
                return c2

            lax.fori_loop(0, _CH, row_body, 0)
            pltpu.sync_copy(msgb, msg_hbm.at[pl.ds(base, _CH)])

        issue_idx(0, 0)
        issue_gather(0, 0)

        def step(k, carry):
            c0 = 2 * k
            c1 = 2 * k + 1

            @pl.when(c1 < n_chunks)
            def _():
                issue_idx(c1, 1)
                issue_gather(c1, 1)
            compute(c0, 0)

            @pl.when(c1 < n_chunks)
            def _():
                @pl.when(c1 + 1 < n_chunks)
                def _():
                    issue_idx(c1 + 1, 0)
                    issue_gather(c1 + 1, 0)
                compute(c1, 1)
            return carry

        lax.fori_loop(0, (n_chunks + 1) // 2, step, 0)

    return pl.kernel(
        body,
        out_type=jax.ShapeDtypeStruct((E_pad, 144), jnp.float32),
        mesh=plsc.VectorSubcoreMesh(core_axis_name="c", subcore_axis_name="s"),
        compiler_params=pltpu.CompilerParams(needs_layout_passes=False,
                                             use_tc_tiling_on_sc=False),
        scratch_types=[
            pltpu.VMEM((1, _CH), jnp.int32),
            pltpu.VMEM((1, _CH), jnp.int32),
            pltpu.VMEM((_CH, 256), jnp.float32),
            pltpu.VMEM((_CH, 128), jnp.float32),
            pltpu.VMEM((1, _CH), jnp.int32),
            pltpu.VMEM((1, _CH), jnp.int32),
            pltpu.VMEM((_CH, 256), jnp.float32),
            pltpu.VMEM((_CH, 128), jnp.float32),
            pltpu.VMEM((_CH, 16), jnp.float32),
            pltpu.VMEM((_CH, 144), jnp.float32),
        ] + [pltpu.SemaphoreType.DMA] * 8,
    )


@functools.lru_cache(maxsize=None)
def _make_pass_b(E_pad, n_chunks_sc, n_sweeps):
    STG = 272

    def body(msg_hbm, dst_hbm, zero_hbm, agg_hbm,
             acc_sh, dstv0, dstv1, eid_st, ldst_st, eidb, ldstb, wbuf,
             sem1, dv0, dv1):
        cid = lax.axis_index("c")
        sid = lax.axis_index("s")
        iota16 = lax.iota(jnp.int32, 16)
        zi = jnp.zeros((16,), jnp.int32)
        junk = jnp.full((16,), _ROWS_SC, jnp.int32)
        dbufs = [(dstv0, dv0), (dstv1, dv1)]
        for t in range(STG // 16):
            eid_st[pl.ds(t * 16, 16)] = zi
            ldst_st[pl.ds(t * 16, 16)] = junk

        def fire_batch():
            for t in range(8):
                sl = pl.ds(t * 16, 16)
                eidb[0, sl] = eid_st[sl]
                ldstb[0, sl] = ldst_st[sl]
            pltpu.async_copy(msg_hbm.at[eidb.at[0]], wbuf, sem1).wait()
            pltpu.sync_copy(wbuf, acc_sh.at[ldstb.at[0]], add=True)

        def cbase(ci):
            return (sid * n_chunks_sc + ci) * _CH

        def issue_dst(ci, p):
            dstv, dv = dbufs[p]
            pltpu.async_copy(dst_hbm.at[pl.ds(cbase(ci), _CH)], dstv.at[0], dv)

        for s in range(n_sweeps):
            lo = (s * 2 + cid) * _ROWS_SC
            pltpu.sync_copy(zero_hbm, acc_sh.at[pl.ds(sid * _ROWS_TILE, _ROWS_TILE)])
            plsc.subcore_barrier()
            issue_dst(0, 0)

            def half_chunk(ci, p, F, lo=lo):
                dstv, dv = dbufs[p]
                base = cbase(ci)
                pltpu.make_async_copy(dst_hbm.at[pl.ds(base, _CH)],
                                      dstv.at[0], dv).wait()

                @pl.when(ci + 1 < n_chunks_sc)
                def _():
                    issue_dst(ci + 1, 1 - p)

                def group_body(g, off):
                    rid = g * 16 + iota16
                    dvec = dstv[0, pl.ds(g * 16, 16)]
                    loc = dvec - lo
                    m = (loc >= 0) & (loc < _ROWS_SC)
                    plsc.store_compressed(eid_st.at[pl.ds(off, 16)],
                                          base + rid, mask=m)
                    plsc.store_compressed(ldst_st.at[pl.ds(off, 16)],
                                          loc, mask=m)
                    return off + jnp.sum(m.astype(jnp.int32))

                F = lax.fori_loop(0, _CH // 16, group_body, F)

                def with_batch(F):
                    fire_batch()
                    for t in range(8):
                        dst_sl = pl.ds(t * 16, 16)
                        src_sl = pl.ds(128 + t * 16, 16)
                        eid_st[dst_sl] = eid_st[src_sl]
                        ldst_st[dst_sl] = ldst_st[src_sl]
                    return F - 128

                return lax.cond(F >= 128, with_batch, lambda F: F, F)

            def chunk_pair(k, F):
                F = half_chunk(2 * k, 0, F)
                return half_chunk(2 * k + 1, 1, F)

            F = lax.fori_loop(0, n_chunks_sc // 2, chunk_pair, jnp.int32(0))
            for t in range(8):
                pos = t * 16 + iota16
                sl = pl.ds(t * 16, 16)
                ldst_st[sl] = jnp.where(pos < F, ldst_st[sl], junk)
            fire_batch()
            plsc.subcore_barrier()
            gbase = s * _SWEEP + cid * _ROWS_SC + sid * _ROWS_TILE
            pltpu.sync_copy(acc_sh.at[pl.ds(sid * _ROWS_TILE, _ROWS_TILE)],
                            agg_hbm.at[pl.ds(gbase, _ROWS_TILE)])
            plsc.subcore_barrier()

    return pl.kernel(
        body,
        out_type=jax.ShapeDtypeStruct((n_sweeps * _SWEEP, 144), jnp.float32),
        mesh=plsc.VectorSubcoreMesh(core_axis_name="c", subcore_axis_name="s"),
        compiler_params=pltpu.CompilerParams(needs_layout_passes=False,
                                             use_tc_tiling_on_sc=False),
        scratch_types=[
            pltpu.VMEM_SHARED((_ROWS_SC + 8, 144), jnp.float32),
            pltpu.VMEM((1, _CH), jnp.int32),
            pltpu.VMEM((1, _CH), jnp.int32),
            pltpu.VMEM((STG,), jnp.int32),
            pltpu.VMEM((STG,), jnp.int32),
            pltpu.VMEM((1, _CH), jnp.int32),
            pltpu.VMEM((1, _CH), jnp.int32),
            pltpu.VMEM((_CH, 144), jnp.float32),
            pltpu.SemaphoreType.DMA,
            pltpu.SemaphoreType.DMA,
            pltpu.SemaphoreType.DMA,
        ],
    )


def _stage3(x, a, rmat, w3, b3, aggs, bds):
    ne = len(aggs)
    N = x.shape[0]
    BN = 400

    def body(x_ref, a_ref, r_ref, w3_ref, b3_ref, *rest):
        agg_refs = rest[:ne]
        bd_refs = rest[ne:2 * ne]
        o_ref = rest[2 * ne]
        R = r_ref[...]
        acc = jnp.zeros((BN, 128), jnp.float32)
        for i in range(ne):
            blk = agg_refs[i][...]
            num = blk[:, :128]
            den = blk[:, 128:136] + 1e-16
            denrep = jnp.dot(den, R, preferred_element_type=jnp.float32)
            acc = acc + jnp.dot(num / denrep, bd_refs[i][...],
                                preferred_element_type=jnp.float32)
        o = (jnp.dot(jax.nn.gelu(acc), w3_ref[...],
                     preferred_element_type=jnp.float32) + b3_ref[0:1, :])
        av = a_ref[0, 0]
        y = av * o + (1.0 - av) * x_ref[...]
        o_ref[...] = jnp.maximum(y, 0.0)

    in_specs = [
        pl.BlockSpec((BN, 128), lambda i: (i, 0)),
        pl.BlockSpec(memory_space=pltpu.SMEM),
        pl.BlockSpec((8, 128), lambda i: (0, 0)),
        pl.BlockSpec((128, 128), lambda i: (0, 0)),
        pl.BlockSpec((8, 128), lambda i: (0, 0)),
    ]
    in_specs += [pl.BlockSpec((BN, 144), lambda i: (i, 0)) for _ in range(ne)]
    in_specs += [pl.BlockSpec((128, 128), lambda i: (0, 0)) for _ in range(ne)]
    return pl.pallas_call(
        body,
        grid=(N // BN,),
        in_specs=in_specs,
        out_specs=pl.BlockSpec((BN, 128), lambda i: (i, 0)),
        out_shape=jax.ShapeDtypeStruct((N, 128), jnp.float32),
    )(x, a, rmat, w3, b3, *aggs, *bds)


def kernel(x_document, x_word, x_medical_concept, x_symptom_category,
           ei_contains, ei_co_occurs, ei_maps_to, ei_belongs_to, ei_related_to,
           ei_rev_contains, ei_rev_maps_to, ei_rev_belongs_to,
           lin_w, lin_b, rel_att, rel_msg, rel_pri, skip):
    xs = [x_document, x_word, x_medical_concept, x_symptom_category]
    eis = [ei_contains, ei_co_occurs, ei_maps_to, ei_belongs_to, ei_related_to,
           ei_rev_contains, ei_rev_maps_to, ei_rev_belongs_to]

    srcs, dsts, epads, nchunks = [], [], [], []
    for e, (s, d, E) in enumerate(_EM):
        ep, nc = _epad(E)
        src = eis[e][0].astype(jnp.int32)
        dst = eis[e][1].astype(jnp.int32)
        srcs.append(jnp.pad(src, (0, ep - E)))
        dsts.append(jnp.pad(dst, (0, ep - E)))
        epads.append(ep)
        nchunks.append(nc)

    eye8 = jnp.eye(8, dtype=jnp.float32)
    rmat = jnp.repeat(eye8, 16, axis=1)
    zero_init = jnp.zeros((_ROWS_TILE, 144), jnp.float32)

    def blockdiag(m):
        return jnp.einsum('hdf,hg->hdgf', m, eye8).reshape(128, 128)

    for l in range(_LAYERS):
        wke, bke, bdrm = {}, {}, {}
        for e, (s, d, E) in enumerate(_EM):
            bd_a = blockdiag(rel_att[l, e] * (rel_pri[l, e] / 4.0)[:, None, None])
            wke[e] = lin_w[l, s, 0] @ bd_a
            bke[e] = lin_b[l, s, 0] @ bd_a
            bdrm[e] = blockdiag(rel_msg[l, e])

        Q, KV = {}, {}
        for t in range(4):
            ws = [lin_w[l, t, 1]]
            bs = [lin_b[l, t, 1]]
            widths = [128]
            for e in _SRC_EDGES[t]:
                ws += [wke[e], lin_w[l, t, 2]]
                bs += [bke[e], lin_b[l, t, 2]]
                widths.append(256)
            wcat = jnp.concatenate(ws, axis=1)
            bcat = jnp.zeros((8, wcat.shape[1]), jnp.float32).at[0].set(
                jnp.concatenate(bs, axis=0))
            outs = _stage1(xs[t], wcat, bcat, tuple(widths))
            Q[t] = outs[0]
            for j, e in enumerate(_SRC_EDGES[t]):
                KV[e] = outs[1 + j]

        MSG, AGG = {}, {}
        for e, (s, d, E) in enumerate(_EM):
            MSG[e] = _make_pass_a(E, epads[e], nchunks[e])(
                KV[e], Q[d], srcs[e], dsts[e])
        for e, (s, d, E) in enumerate(_EM):
            n_sweeps = -(-_N_NODES[d] // _SWEEP)
            n_chunks_sc = epads[e] // (16 * _CH)
            AGG[e] = _make_pass_b(epads[e], n_chunks_sc, n_sweeps)(
                MSG[e], dsts[e], zero_init)

        new_xs = []
        for t in range(4):
            a = jax.nn.sigmoid(skip[l, t]).reshape(1, 1).astype(jnp.float32)
            b3 = jnp.zeros((8, 128), jnp.float32).at[0].set(lin_b[l, t, 3])
            aggs = [AGG[e] for e in _IN_EDGES[t]]
            bds = [bdrm[e] for e in _IN_EDGES[t]]
            new_xs.append(_stage3(xs[t], a, rmat, lin_w[l, t, 3], b3, aggs, bds))
        xs = new_xs

    return tuple(xs)


# speedup vs baseline: 15.3807x; 1.0126x over previous
import functools

import jax
import jax.numpy as jnp
from jax import lax
from jax.experimental import pallas as pl
from jax.experimental.pallas import tpu as pltpu
from jax.experimental.pallas import tpu_sc as plsc

_N_NODES = [10000, 100000, 50000, 10000]
_EM = [(0, 1, 100000), (1, 1, 100000), (1, 2, 50000), (2, 3, 20000),
       (2, 2, 50000), (1, 0, 100000), (2, 1, 50000), (3, 2, 20000)]
_HID, _HEADS, _DH, _LAYERS = 128, 8, 16, 2
_CH = 128
_ROWS_SC = 10080
_ROWS_TILE = _ROWS_SC // 16
_SWEEP = 2 * _ROWS_SC

_SRC_EDGES = {0: [0], 1: [1, 2, 5], 2: [3, 4, 6], 3: [7]}
_IN_EDGES = {0: [5], 1: [0, 1, 6], 2: [2, 4, 7], 3: [3]}


def _epad(E):
    n_chunks = -(-E // (32 * _CH))
    return 32 * _CH * n_chunks, n_chunks


def _stage1(x, wcat, bcat, widths):
    N = x.shape[0]
    BN = 400
    offs = [0]
    for w in widths:
        offs.append(offs[-1] + w)
    W = offs[-1]

    def body(x_ref, w_ref, b_ref, *o_refs):
        xv = x_ref[...]
        for k in range(len(widths)):
            o_refs[k][...] = (
                jnp.dot(xv, w_ref[:, offs[k]:offs[k + 1]],
                        preferred_element_type=jnp.float32)
                + b_ref[0:1, offs[k]:offs[k + 1]])

    return pl.pallas_call(
        body,
        grid=(N // BN,),
        in_specs=[
            pl.BlockSpec((BN, 128), lambda i: (i, 0)),
            pl.BlockSpec((128, W), lambda i: (0, 0)),
            pl.BlockSpec((8, W), lambda i: (0, 0)),
        ],
        out_specs=[pl.BlockSpec((BN, w), lambda i: (i, 0)) for w in widths],
        out_shape=[jax.ShapeDtypeStruct((N, w), jnp.float32) for w in widths],
    )(x, wcat, bcat)


@functools.lru_cache(maxsize=None)
def _make_pass_a(plan, dst_types):
    def body(*refs):
        kvs = refs[0:8]
        qs = refs[8:12]
        srcs = refs[12:20]
        dsts = refs[20:28]
        msgs = refs[28:36]
        (srcv0, dstv0, kvrows0, qrows0, srcv1, dstv1, kvrows1, qrows1,
         atb, msgb) = refs[36:46]
        sems = refs[46:54]
        cid = lax.axis_index("c")
        sid = lax.axis_index("s")
        wid = sid * 2 + cid
        iota16 = lax.iota(jnp.int32, 16)
        bufs = [(srcv0, dstv0, kvrows0, qrows0, sems[0:4]),
                (srcv1, dstv1, kvrows1, qrows1, sems[4:8])]

        for e in range(8):
            _pa_section(kvs[e], qs[dst_types[e]], srcs[e], dsts[e], msgs[e],
                        plan[e], wid, iota16, bufs, atb, msgb)

    def _pa_section(kv_hbm, q_hbm, src_hbm, dst_hbm, msg_hbm,
                    eplan, wid, iota16, bufs, atb, msgb):
        E, E_pad, n_chunks = eplan

        def cbase(ci):
            return (wid * n_chunks + ci) * _CH

        def issue_idx(ci, p):
            srcv, dstv, _, _, (si, di, _, _) = bufs[p]
            pltpu.async_copy(src_hbm.at[pl.ds(cbase(ci), _CH)], srcv.at[0], si)
            pltpu.async_copy(dst_hbm.at[pl.ds(cbase(ci), _CH)], dstv.at[0], di)

        def issue_gather(ci, p):
            srcv, dstv, kvrows, qrows, (si, di, gk, gq) = bufs[p]
            pltpu.make_async_copy(src_hbm.at[pl.ds(cbase(ci), _CH)],
                                  srcv.at[0], si).wait()
            pltpu.make_async_copy(dst_hbm.at[pl.ds(cbase(ci), _CH)],
                                  dstv.at[0], di).wait()
            pltpu.async_copy(kv_hbm.at[srcv.at[0]], kvrows, gk)
            pltpu.async_copy(q_hbm.at[dstv.at[0]], qrows, gq)

        def compute(ci, p):
            srcv, dstv, kvrows, qrows, (si, di, gk, gq) = bufs[p]
            base = cbase(ci)
            pltpu.make_async_copy(kv_hbm.at[srcv.at[0]], kvrows, gk).wait()
            pltpu.make_async_copy(q_hbm.at[dstv.at[0]], qrows, gq).wait()

            def group_body(g, c2):
                rid = g * 16 + iota16
                valid = (base + rid) < E

                def head_body(h, c3):
                    acc = jnp.zeros((16,), jnp.float32)
                    for j in range(16):
                        col = jnp.full((16,), h * 16 + j, jnp.int32)
                        acc = acc + (plsc.load_gather(qrows, [rid, col])
                                     * plsc.load_gather(kvrows, [rid, col]))
                    ex = jnp.where(valid, jnp.exp(acc), 0.0)
                    plsc.store_scatter(atb, [rid, jnp.full((16,), h, jnp.int32)], ex)
                    return c3

                lax.fori_loop(0, 8, head_body, 0)
                return c2

            lax.fori_loop(0, _CH // 16, group_body, 0)

            def row_body(r, c2):
                rv = jnp.full((16,), r, jnp.int32)
                av = plsc.load_gather(atb, [rv, iota16])
                av = jnp.where(iota16 < 8, av, 0.0)
                plsc.store_scatter(msgb, [rv, 128 + iota16], av)

                def hrow_body(h, c3):
                    sc = plsc.load_gather(atb, [rv, jnp.full((16,), h, jnp.int32)])
                    vv = plsc.load_gather(kvrows, [rv, 128 + h * 16 + iota16])
                    plsc.store_scatter(msgb, [rv, h * 16 + iota16], vv * sc)
                    return c3

                lax.fori_loop(0, 8, hrow_body, 0)
                return c2

            lax.fori_loop(0, _CH, row_body, 0)
            pltpu.sync_copy(msgb, msg_hbm.at[pl.ds(base, _CH)])

        issue_idx(0, 0)
        issue_gather(0, 0)

        def step(k, carry):
            c0 = 2 * k
            c1 = 2 * k + 1

            @pl.when(c1 < n_chunks)
            def _():
                issue_idx(c1, 1)
                issue_gather(c1, 1)
            compute(c0, 0)

            @pl.when(c1 < n_chunks)
            def _():
                @pl.when(c1 + 1 < n_chunks)
                def _():
                    issue_idx(c1 + 1, 0)
                    issue_gather(c1 + 1, 0)
                compute(c1, 1)
            return carry

        lax.fori_loop(0, (n_chunks + 1) // 2, step, 0)

    return pl.kernel(
        body,
        out_type=[jax.ShapeDtypeStruct((ep[1], 144), jnp.float32) for ep in plan],
        mesh=plsc.VectorSubcoreMesh(core_axis_name="c", subcore_axis_name="s"),
        compiler_params=pltpu.CompilerParams(needs_layout_passes=False,
                                             use_tc_tiling_on_sc=False),
        scratch_types=[
            pltpu.VMEM((1, _CH), jnp.int32),
            pltpu.VMEM((1, _CH), jnp.int32),
            pltpu.VMEM((_CH, 256), jnp.float32),
            pltpu.VMEM((_CH, 128), jnp.float32),
            pltpu.VMEM((1, _CH), jnp.int32),
            pltpu.VMEM((1, _CH), jnp.int32),
            pltpu.VMEM((_CH, 256), jnp.float32),
            pltpu.VMEM((_CH, 128), jnp.float32),
            pltpu.VMEM((_CH, 16), jnp.float32),
            pltpu.VMEM((_CH, 144), jnp.float32),
        ] + [pltpu.SemaphoreType.DMA] * 8,
    )


@functools.lru_cache(maxsize=None)
def _make_pass_b(planb):
    STG = 272

    def body(*refs):
        msgs = refs[0:8]
        dsts = refs[8:16]
        zero_hbm = refs[16]
        aggs = refs[17:25]
        acc_sh, dstv0, dstv1, eid_st, ldst_st, eidb, ldstb, wbuf = refs[25:33]
        sem1, dv0, dv1 = refs[33:36]
        cid = lax.axis_index("c")
        sid = lax.axis_index("s")
        iota16 = lax.iota(jnp.int32, 16)
        zi = jnp.zeros((16,), jnp.int32)
        junk = jnp.full((16,), _ROWS_SC, jnp.int32)
        dbufs = [(dstv0, dv0), (dstv1, dv1)]
        for e in range(8):
            _pb_section(msgs[e], dsts[e], zero_hbm, aggs[e], planb[e],
                        cid, sid, iota16, zi, junk, dbufs,
                        acc_sh, eid_st, ldst_st, eidb, ldstb, wbuf, sem1)

    def _pb_section(msg_hbm, dst_hbm, zero_hbm, agg_hbm, eplan,
                    cid, sid, iota16, zi, junk, dbufs,
                    acc_sh, eid_st, ldst_st, eidb, ldstb, wbuf, sem1):
        E_pad, n_chunks_sc, n_sweeps = eplan

        def init_body(t, c):
            eid_st[pl.ds(t * 16, 16)] = zi
            ldst_st[pl.ds(t * 16, 16)] = junk
            return c

        lax.fori_loop(0, STG // 16, init_body, 0)

        def fire_batch():
            def cp_body(t, c):
                sl = pl.ds(t * 16, 16)
                eidb[0, sl] = eid_st[sl]
                ldstb[0, sl] = ldst_st[sl]
                return c

            lax.fori_loop(0, 8, cp_body, 0)
            pltpu.async_copy(msg_hbm.at[eidb.at[0]], wbuf, sem1).wait()
            pltpu.sync_copy(wbuf, acc_sh.at[ldstb.at[0]], add=True)

        def cbase(ci):
            return (sid * n_chunks_sc + ci) * _CH

        def issue_dst(ci, p):
            dstv, dv = dbufs[p]
            pltpu.async_copy(dst_hbm.at[pl.ds(cbase(ci), _CH)], dstv.at[0], dv)

        for s in range(n_sweeps):
            lo = (s * 2 + cid) * _ROWS_SC
            pltpu.sync_copy(zero_hbm, acc_sh.at[pl.ds(sid * _ROWS_TILE, _ROWS_TILE)])
            plsc.subcore_barrier()
            issue_dst(0, 0)

            def half_chunk(ci, p, F, lo=lo):
                dstv, dv = dbufs[p]
                base = cbase(ci)
                pltpu.make_async_copy(dst_hbm.at[pl.ds(base, _CH)],
                                      dstv.at[0], dv).wait()

                @pl.when(ci + 1 < n_chunks_sc)
                def _():
                    issue_dst(ci + 1, 1 - p)

                def group_body(g, off):
                    rid = g * 16 + iota16
                    dvec = dstv[0, pl.ds(g * 16, 16)]
                    loc = dvec - lo
                    m = (loc >= 0) & (loc < _ROWS_SC)
                    plsc.store_compressed(eid_st.at[pl.ds(off, 16)],
                                          base + rid, mask=m)
                    plsc.store_compressed(ldst_st.at[pl.ds(off, 16)],
                                          loc, mask=m)
                    return off + jnp.sum(m.astype(jnp.int32))

                F = lax.fori_loop(0, _CH // 16, group_body, F)

                def with_batch(F):
                    fire_batch()

                    def shift_body(t, c):
                        dst_sl = pl.ds(t * 16, 16)
                        src_sl = pl.ds(128 + t * 16, 16)
                        eid_st[dst_sl] = eid_st[src_sl]
                        ldst_st[dst_sl] = ldst_st[src_sl]
                        return c

                    lax.fori_loop(0, 8, shift_body, 0)
                    return F - 128

                return lax.cond(F >= 128, with_batch, lambda F: F, F)

            def chunk_pair(k, F):
                F = half_chunk(2 * k, 0, F)
                return half_chunk(2 * k + 1, 1, F)

            F = lax.fori_loop(0, n_chunks_sc // 2, chunk_pair, jnp.int32(0))

            def flush_body(t, c):
                pos = t * 16 + iota16
                sl = pl.ds(t * 16, 16)
                ldst_st[sl] = jnp.where(pos < F, ldst_st[sl], junk)
                return c

            lax.fori_loop(0, 8, flush_body, 0)
            fire_batch()
            plsc.subcore_barrier()
            gbase = s * _SWEEP + cid * _ROWS_SC + sid * _ROWS_TILE
            pltpu.sync_copy(acc_sh.at[pl.ds(sid * _ROWS_TILE, _ROWS_TILE)],
                            agg_hbm.at[pl.ds(gbase, _ROWS_TILE)])
            plsc.subcore_barrier()

    return pl.kernel(
        body,
        out_type=[jax.ShapeDtypeStruct((ep[2] * _SWEEP, 144), jnp.float32)
                  for ep in planb],
        mesh=plsc.VectorSubcoreMesh(core_axis_name="c", subcore_axis_name="s"),
        compiler_params=pltpu.CompilerParams(needs_layout_passes=False,
                                             use_tc_tiling_on_sc=False),
        scratch_types=[
            pltpu.VMEM_SHARED((_ROWS_SC + 8, 144), jnp.float32),
            pltpu.VMEM((1, _CH), jnp.int32),
            pltpu.VMEM((1, _CH), jnp.int32),
            pltpu.VMEM((STG,), jnp.int32),
            pltpu.VMEM((STG,), jnp.int32),
            pltpu.VMEM((1, _CH), jnp.int32),
            pltpu.VMEM((1, _CH), jnp.int32),
            pltpu.VMEM((_CH, 144), jnp.float32),
            pltpu.SemaphoreType.DMA,
            pltpu.SemaphoreType.DMA,
            pltpu.SemaphoreType.DMA,
        ],
    )


def _stage3(x, a, rmat, w3, b3, aggs, bds):
    ne = len(aggs)
    N = x.shape[0]
    BN = 400

    def body(x_ref, a_ref, r_ref, w3_ref, b3_ref, *rest):
        agg_refs = rest[:ne]
        bd_refs = rest[ne:2 * ne]
        o_ref = rest[2 * ne]
        R = r_ref[...]
        acc = jnp.zeros((BN, 128), jnp.float32)
        for i in range(ne):
            blk = agg_refs[i][...]
            num = blk[:, :128]
            den = blk[:, 128:136] + 1e-16
            denrep = jnp.dot(den, R, preferred_element_type=jnp.float32)
            acc = acc + jnp.dot(num / denrep, bd_refs[i][...],
                                preferred_element_type=jnp.float32)
        o = (jnp.dot(jax.nn.gelu(acc), w3_ref[...],
                     preferred_element_type=jnp.float32) + b3_ref[0:1, :])
        av = a_ref[0, 0]
        y = av * o + (1.0 - av) * x_ref[...]
        o_ref[...] = jnp.maximum(y, 0.0)

    in_specs = [
        pl.BlockSpec((BN, 128), lambda i: (i, 0)),
        pl.BlockSpec(memory_space=pltpu.SMEM),
        pl.BlockSpec((8, 128), lambda i: (0, 0)),
        pl.BlockSpec((128, 128), lambda i: (0, 0)),
        pl.BlockSpec((8, 128), lambda i: (0, 0)),
    ]
    in_specs += [pl.BlockSpec((BN, 144), lambda i: (i, 0)) for _ in range(ne)]
    in_specs += [pl.BlockSpec((128, 128), lambda i: (0, 0)) for _ in range(ne)]
    return pl.pallas_call(
        body,
        grid=(N // BN,),
        in_specs=in_specs,
        out_specs=pl.BlockSpec((BN, 128), lambda i: (i, 0)),
        out_shape=jax.ShapeDtypeStruct((N, 128), jnp.float32),
    )(x, a, rmat, w3, b3, *aggs, *bds)


def kernel(x_document, x_word, x_medical_concept, x_symptom_category,
           ei_contains, ei_co_occurs, ei_maps_to, ei_belongs_to, ei_related_to,
           ei_rev_contains, ei_rev_maps_to, ei_rev_belongs_to,
           lin_w, lin_b, rel_att, rel_msg, rel_pri, skip):
    xs = [x_document, x_word, x_medical_concept, x_symptom_category]
    eis = [ei_contains, ei_co_occurs, ei_maps_to, ei_belongs_to, ei_related_to,
           ei_rev_contains, ei_rev_maps_to, ei_rev_belongs_to]

    srcs, dsts, epads, nchunks = [], [], [], []
    for e, (s, d, E) in enumerate(_EM):
        ep, nc = _epad(E)
        src = eis[e][0].astype(jnp.int32)
        dst = eis[e][1].astype(jnp.int32)
        srcs.append(jnp.pad(src, (0, ep - E)))
        dsts.append(jnp.pad(dst, (0, ep - E)))
        epads.append(ep)
        nchunks.append(nc)

    eye8 = jnp.eye(8, dtype=jnp.float32)
    rmat = jnp.repeat(eye8, 16, axis=1)
    zero_init = jnp.zeros((_ROWS_TILE, 144), jnp.float32)

    def blockdiag(m):
        return jnp.einsum('hdf,hg->hdgf', m, eye8).reshape(128, 128)

    for l in range(_LAYERS):
        wke, bke, bdrm = {}, {}, {}
        for e, (s, d, E) in enumerate(_EM):
            bd_a = blockdiag(rel_att[l, e] * (rel_pri[l, e] / 4.0)[:, None, None])
            wke[e] = lin_w[l, s, 0] @ bd_a
            bke[e] = lin_b[l, s, 0] @ bd_a
            bdrm[e] = blockdiag(rel_msg[l, e])

        Q, KV = {}, {}
        for t in range(4):
            ws = [lin_w[l, t, 1]]
            bs = [lin_b[l, t, 1]]
            widths = [128]
            for e in _SRC_EDGES[t]:
                ws += [wke[e], lin_w[l, t, 2]]
                bs += [bke[e], lin_b[l, t, 2]]
                widths.append(256)
            wcat = jnp.concatenate(ws, axis=1)
            bcat = jnp.zeros((8, wcat.shape[1]), jnp.float32).at[0].set(
                jnp.concatenate(bs, axis=0))
            outs = _stage1(xs[t], wcat, bcat, tuple(widths))
            Q[t] = outs[0]
            for j, e in enumerate(_SRC_EDGES[t]):
                KV[e] = outs[1 + j]

        plan = tuple((E, epads[e], nchunks[e]) for e, (s, d, E) in enumerate(_EM))
        dst_types = tuple(d for (s, d, E) in _EM)
        MSG = _make_pass_a(plan, dst_types)(
            *[KV[e] for e in range(8)], *[Q[t] for t in range(4)],
            *srcs, *dsts)
        planb = tuple((epads[e], epads[e] // (16 * _CH),
                       -(-_N_NODES[d] // _SWEEP))
                      for e, (s, d, E) in enumerate(_EM))
        AGG = _make_pass_b(planb)(*MSG, *dsts, zero_init)

        new_xs = []
        for t in range(4):
            a = jax.nn.sigmoid(skip[l, t]).reshape(1, 1).astype(jnp.float32)
            b3 = jnp.zeros((8, 128), jnp.float32).at[0].set(lin_b[l, t, 3])
            aggs = [AGG[e] for e in _IN_EDGES[t]]
            bds = [bdrm[e] for e in _IN_EDGES[t]]
            new_xs.append(_stage3(xs[t], a, rmat, lin_w[l, t, 3], b3, aggs, bds))
        xs = new_xs

    return tuple(xs)


# speedup vs baseline: 19.3329x; 1.2570x over previous
import functools

import jax
import jax.numpy as jnp
from jax import lax
from jax.experimental import pallas as pl
from jax.experimental.pallas import tpu as pltpu
from jax.experimental.pallas import tpu_sc as plsc

_N_NODES = [10000, 100000, 50000, 10000]
_EM = [(0, 1, 100000), (1, 1, 100000), (1, 2, 50000), (2, 3, 20000),
       (2, 2, 50000), (1, 0, 100000), (2, 1, 50000), (3, 2, 20000)]
_HID, _HEADS, _DH, _LAYERS = 128, 8, 16, 2
_CH = 128
_ROWS_SC = 10080
_ROWS_TILE = _ROWS_SC // 16
_SWEEP = 2 * _ROWS_SC

_SRC_EDGES = {0: [0], 1: [1, 2, 5], 2: [3, 4, 6], 3: [7]}
_IN_EDGES = {0: [5], 1: [0, 1, 6], 2: [2, 4, 7], 3: [3]}


def _epad(E):
    n_chunks = -(-E // (32 * _CH))
    return 32 * _CH * n_chunks, n_chunks


def _stage1(x, wcat, bcat, widths):
    N = x.shape[0]
    BN = 400
    offs = [0]
    for w in widths:
        offs.append(offs[-1] + w)
    W = offs[-1]

    def body(x_ref, w_ref, b_ref, *o_refs):
        xv = x_ref[...]
        for k in range(len(widths)):
            o_refs[k][...] = (
                jnp.dot(xv, w_ref[:, offs[k]:offs[k + 1]],
                        preferred_element_type=jnp.float32)
                + b_ref[0:1, offs[k]:offs[k + 1]])

    return pl.pallas_call(
        body,
        grid=(N // BN,),
        in_specs=[
            pl.BlockSpec((BN, 128), lambda i: (i, 0)),
            pl.BlockSpec((128, W), lambda i: (0, 0)),
            pl.BlockSpec((8, W), lambda i: (0, 0)),
        ],
        out_specs=[pl.BlockSpec((BN, w), lambda i: (i, 0)) for w in widths],
        out_shape=[jax.ShapeDtypeStruct((N, w), jnp.float32) for w in widths],
    )(x, wcat, bcat)


@functools.lru_cache(maxsize=None)
def _make_pass_a(plan, dst_types):
    def body(*refs):
        kvs = refs[0:8]
        qs = refs[8:12]
        srcs = refs[12:20]
        dsts = refs[20:28]
        msgs = refs[28:36]
        (srcv0, dstv0, kvrows0, qrows0, srcv1, dstv1, kvrows1, qrows1,
         atb, msgb) = refs[36:46]
        sems = refs[46:54]
        cid = lax.axis_index("c")
        sid = lax.axis_index("s")
        wid = sid * 2 + cid
        iota16 = lax.iota(jnp.int32, 16)
        bufs = [(srcv0, dstv0, kvrows0, qrows0, sems[0:4]),
                (srcv1, dstv1, kvrows1, qrows1, sems[4:8])]

        for e in range(8):
            _pa_section(kvs[e], qs[dst_types[e]], srcs[e], dsts[e], msgs[e],
                        plan[e], wid, iota16, bufs, atb, msgb)

    def _pa_section(kv_hbm, q_hbm, src_hbm, dst_hbm, msg_hbm,
                    eplan, wid, iota16, bufs, atb, msgb):
        E, E_pad, n_chunks = eplan

        def cbase(ci):
            return (wid * n_chunks + ci) * _CH

        def issue_idx(ci, p):
            srcv, dstv, _, _, (si, di, _, _) = bufs[p]
            pltpu.async_copy(src_hbm.at[pl.ds(cbase(ci), _CH)], srcv.at[0], si)
            pltpu.async_copy(dst_hbm.at[pl.ds(cbase(ci), _CH)], dstv.at[0], di)

        def issue_gather(ci, p):
            srcv, dstv, kvrows, qrows, (si, di, gk, gq) = bufs[p]
            pltpu.make_async_copy(src_hbm.at[pl.ds(cbase(ci), _CH)],
                                  srcv.at[0], si).wait()
            pltpu.make_async_copy(dst_hbm.at[pl.ds(cbase(ci), _CH)],
                                  dstv.at[0], di).wait()
            pltpu.async_copy(kv_hbm.at[srcv.at[0]], kvrows, gk)
            pltpu.async_copy(q_hbm.at[dstv.at[0]], qrows, gq)

        def compute(ci, p):
            srcv, dstv, kvrows, qrows, (si, di, gk, gq) = bufs[p]
            base = cbase(ci)
            pltpu.make_async_copy(kv_hbm.at[srcv.at[0]], kvrows, gk).wait()
            pltpu.make_async_copy(q_hbm.at[dstv.at[0]], qrows, gq).wait()

            def group_body(g, c2):
                rid = g * 16 + iota16
                valid = (base + rid) < E

                def head_body(h, c3):
                    acc = jnp.zeros((16,), jnp.float32)
                    for j in range(16):
                        col = h * 16 + ((iota16 + j) & 15)
                        acc = acc + (plsc.load_gather(qrows, [rid, col])
                                     * plsc.load_gather(kvrows, [rid, col]))
                    ex = jnp.where(valid, jnp.exp(acc), 0.0)
                    plsc.store_scatter(atb, [rid, jnp.full((16,), h, jnp.int32)], ex)
                    return c3

                lax.fori_loop(0, 8, head_body, 0)
                return c2

            lax.fori_loop(0, _CH // 16, group_body, 0)

            def row_body(r, c2):
                rv = jnp.full((16,), r, jnp.int32)
                av = plsc.load_gather(atb, [rv, iota16])
                av = jnp.where(iota16 < 8, av, 0.0)
                plsc.store_scatter(msgb, [rv, 128 + iota16], av)

                def hrow_body(h, c3):
                    sc = plsc.load_gather(atb, [rv, jnp.full((16,), h, jnp.int32)])
                    vv = plsc.load_gather(kvrows, [rv, 128 + h * 16 + iota16])
                    plsc.store_scatter(msgb, [rv, h * 16 + iota16], vv * sc)
                    return c3

                lax.fori_loop(0, 8, hrow_body, 0)
                return c2

            lax.fori_loop(0, _CH, row_body, 0)
            pltpu.sync_copy(msgb, msg_hbm.at[pl.ds(base, _CH)])

        issue_idx(0, 0)
        issue_gather(0, 0)

        def step(k, carry):
            c0 = 2 * k
            c1 = 2 * k + 1

            @pl.when(c1 < n_chunks)
            def _():
                issue_idx(c1, 1)
                issue_gather(c1, 1)
            compute(c0, 0)

            @pl.when(c1 < n_chunks)
            def _():
                @pl.when(c1 + 1 < n_chunks)
                def _():
                    issue_idx(c1 + 1, 0)
                    issue_gather(c1 + 1, 0)
                compute(c1, 1)
            return carry

        lax.fori_loop(0, (n_chunks + 1) // 2, step, 0)

    return pl.kernel(
        body,
        out_type=[jax.ShapeDtypeStruct((ep[1], 144), jnp.float32) for ep in plan],
        mesh=plsc.VectorSubcoreMesh(core_axis_name="c", subcore_axis_name="s"),
        compiler_params=pltpu.CompilerParams(needs_layout_passes=False,
                                             use_tc_tiling_on_sc=False),
        scratch_types=[
            pltpu.VMEM((1, _CH), jnp.int32),
            pltpu.VMEM((1, _CH), jnp.int32),
            pltpu.VMEM((_CH, 256), jnp.float32),
            pltpu.VMEM((_CH, 128), jnp.float32),
            pltpu.VMEM((1, _CH), jnp.int32),
            pltpu.VMEM((1, _CH), jnp.int32),
            pltpu.VMEM((_CH, 256), jnp.float32),
            pltpu.VMEM((_CH, 128), jnp.float32),
            pltpu.VMEM((_CH, 16), jnp.float32),
            pltpu.VMEM((_CH, 144), jnp.float32),
        ] + [pltpu.SemaphoreType.DMA] * 8,
    )


@functools.lru_cache(maxsize=None)
def _make_pass_b(planb):
    STG = 272

    def body(*refs):
        msgs = refs[0:8]
        dsts = refs[8:16]
        zero_hbm = refs[16]
        aggs = refs[17:25]
        acc_sh, dstv0, dstv1, eid_st, ldst_st, eidb, ldstb, wbuf = refs[25:33]
        sem1, dv0, dv1 = refs[33:36]
        cid = lax.axis_index("c")
        sid = lax.axis_index("s")
        iota16 = lax.iota(jnp.int32, 16)
        zi = jnp.zeros((16,), jnp.int32)
        junk = jnp.full((16,), _ROWS_SC, jnp.int32)
        dbufs = [(dstv0, dv0), (dstv1, dv1)]
        for e in range(8):
            _pb_section(msgs[e], dsts[e], zero_hbm, aggs[e], planb[e],
                        cid, sid, iota16, zi, junk, dbufs,
                        acc_sh, eid_st, ldst_st, eidb, ldstb, wbuf, sem1)

    def _pb_section(msg_hbm, dst_hbm, zero_hbm, agg_hbm, eplan,
                    cid, sid, iota16, zi, junk, dbufs,
                    acc_sh, eid_st, ldst_st, eidb, ldstb, wbuf, sem1):
        E_pad, n_chunks_sc, n_sweeps = eplan

        def init_body(t, c):
            eid_st[pl.ds(t * 16, 16)] = zi
            ldst_st[pl.ds(t * 16, 16)] = junk
            return c

        lax.fori_loop(0, STG // 16, init_body, 0)

        def fire_batch():
            def cp_body(t, c):
                sl = pl.ds(t * 16, 16)
                eidb[0, sl] = eid_st[sl]
                ldstb[0, sl] = ldst_st[sl]
                return c

            lax.fori_loop(0, 8, cp_body, 0)
            pltpu.async_copy(msg_hbm.at[eidb.at[0]], wbuf, sem1).wait()
            pltpu.sync_copy(wbuf, acc_sh.at[ldstb.at[0]], add=True)

        def cbase(ci):
            return (sid * n_chunks_sc + ci) * _CH

        def issue_dst(ci, p):
            dstv, dv = dbufs[p]
            pltpu.async_copy(dst_hbm.at[pl.ds(cbase(ci), _CH)], dstv.at[0], dv)

        for s in range(n_sweeps):
            lo = (s * 2 + cid) * _ROWS_SC
            pltpu.sync_copy(zero_hbm, acc_sh.at[pl.ds(sid * _ROWS_TILE, _ROWS_TILE)])
            plsc.subcore_barrier()
            issue_dst(0, 0)

            def half_chunk(ci, p, F, lo=lo):
                dstv, dv = dbufs[p]
                base = cbase(ci)
                pltpu.make_async_copy(dst_hbm.at[pl.ds(base, _CH)],
                                      dstv.at[0], dv).wait()

                @pl.when(ci + 1 < n_chunks_sc)
                def _():
                    issue_dst(ci + 1, 1 - p)

                def group_body(g, off):
                    rid = g * 16 + iota16
                    dvec = dstv[0, pl.ds(g * 16, 16)]
                    loc = dvec - lo
                    m = (loc >= 0) & (loc < _ROWS_SC)
                    plsc.store_compressed(eid_st.at[pl.ds(off, 16)],
                                          base + rid, mask=m)
                    plsc.store_compressed(ldst_st.at[pl.ds(off, 16)],
                                          loc, mask=m)
                    return off + jnp.sum(m.astype(jnp.int32))

                F = lax.fori_loop(0, _CH // 16, group_body, F)

                def with_batch(F):
                    fire_batch()

                    def shift_body(t, c):
                        dst_sl = pl.ds(t * 16, 16)
                        src_sl = pl.ds(128 + t * 16, 16)
                        eid_st[dst_sl] = eid_st[src_sl]
                        ldst_st[dst_sl] = ldst_st[src_sl]
                        return c

                    lax.fori_loop(0, 8, shift_body, 0)
                    return F - 128

                return lax.cond(F >= 128, with_batch, lambda F: F, F)

            def chunk_pair(k, F):
                F = half_chunk(2 * k, 0, F)
                return half_chunk(2 * k + 1, 1, F)

            F = lax.fori_loop(0, n_chunks_sc // 2, chunk_pair, jnp.int32(0))

            def flush_body(t, c):
                pos = t * 16 + iota16
                sl = pl.ds(t * 16, 16)
                ldst_st[sl] = jnp.where(pos < F, ldst_st[sl], junk)
                return c

            lax.fori_loop(0, 8, flush_body, 0)
            fire_batch()
            plsc.subcore_barrier()
            gbase = s * _SWEEP + cid * _ROWS_SC + sid * _ROWS_TILE
            pltpu.sync_copy(acc_sh.at[pl.ds(sid * _ROWS_TILE, _ROWS_TILE)],
                            agg_hbm.at[pl.ds(gbase, _ROWS_TILE)])
            plsc.subcore_barrier()

    return pl.kernel(
        body,
        out_type=[jax.ShapeDtypeStruct((ep[2] * _SWEEP, 144), jnp.float32)
                  for ep in planb],
        mesh=plsc.VectorSubcoreMesh(core_axis_name="c", subcore_axis_name="s"),
        compiler_params=pltpu.CompilerParams(needs_layout_passes=False,
                                             use_tc_tiling_on_sc=False),
        scratch_types=[
            pltpu.VMEM_SHARED((_ROWS_SC + 8, 144), jnp.float32),
            pltpu.VMEM((1, _CH), jnp.int32),
            pltpu.VMEM((1, _CH), jnp.int32),
            pltpu.VMEM((STG,), jnp.int32),
            pltpu.VMEM((STG,), jnp.int32),
            pltpu.VMEM((1, _CH), jnp.int32),
            pltpu.VMEM((1, _CH), jnp.int32),
            pltpu.VMEM((_CH, 144), jnp.float32),
            pltpu.SemaphoreType.DMA,
            pltpu.SemaphoreType.DMA,
            pltpu.SemaphoreType.DMA,
        ],
    )


def _stage3(x, a, rmat, w3, b3, aggs, bds):
    ne = len(aggs)
    N = x.shape[0]
    BN = 400

    def body(x_ref, a_ref, r_ref, w3_ref, b3_ref, *rest):
        agg_refs = rest[:ne]
        bd_refs = rest[ne:2 * ne]
        o_ref = rest[2 * ne]
        R = r_ref[...]
        acc = jnp.zeros((BN, 128), jnp.float32)
        for i in range(ne):
            blk = agg_refs[i][...]
            num = blk[:, :128]
            den = blk[:, 128:136] + 1e-16
            denrep = jnp.dot(den, R, preferred_element_type=jnp.float32)
            acc = acc + jnp.dot(num / denrep, bd_refs[i][...],
                                preferred_element_type=jnp.float32)
        o = (jnp.dot(jax.nn.gelu(acc), w3_ref[...],
                     preferred_element_type=jnp.float32) + b3_ref[0:1, :])
        av = a_ref[0, 0]
        y = av * o + (1.0 - av) * x_ref[...]
        o_ref[...] = jnp.maximum(y, 0.0)

    in_specs = [
        pl.BlockSpec((BN, 128), lambda i: (i, 0)),
        pl.BlockSpec(memory_space=pltpu.SMEM),
        pl.BlockSpec((8, 128), lambda i: (0, 0)),
        pl.BlockSpec((128, 128), lambda i: (0, 0)),
        pl.BlockSpec((8, 128), lambda i: (0, 0)),
    ]
    in_specs += [pl.BlockSpec((BN, 144), lambda i: (i, 0)) for _ in range(ne)]
    in_specs += [pl.BlockSpec((128, 128), lambda i: (0, 0)) for _ in range(ne)]
    return pl.pallas_call(
        body,
        grid=(N // BN,),
        in_specs=in_specs,
        out_specs=pl.BlockSpec((BN, 128), lambda i: (i, 0)),
        out_shape=jax.ShapeDtypeStruct((N, 128), jnp.float32),
    )(x, a, rmat, w3, b3, *aggs, *bds)


def kernel(x_document, x_word, x_medical_concept, x_symptom_category,
           ei_contains, ei_co_occurs, ei_maps_to, ei_belongs_to, ei_related_to,
           ei_rev_contains, ei_rev_maps_to, ei_rev_belongs_to,
           lin_w, lin_b, rel_att, rel_msg, rel_pri, skip):
    xs = [x_document, x_word, x_medical_concept, x_symptom_category]
    eis = [ei_contains, ei_co_occurs, ei_maps_to, ei_belongs_to, ei_related_to,
           ei_rev_contains, ei_rev_maps_to, ei_rev_belongs_to]

    srcs, dsts, epads, nchunks = [], [], [], []
    for e, (s, d, E) in enumerate(_EM):
        ep, nc = _epad(E)
        src = eis[e][0].astype(jnp.int32)
        dst = eis[e][1].astype(jnp.int32)
        srcs.append(jnp.pad(src, (0, ep - E)))
        dsts.append(jnp.pad(dst, (0, ep - E)))
        epads.append(ep)
        nchunks.append(nc)

    eye8 = jnp.eye(8, dtype=jnp.float32)
    rmat = jnp.repeat(eye8, 16, axis=1)
    zero_init = jnp.zeros((_ROWS_TILE, 144), jnp.float32)

    def blockdiag(m):
        return jnp.einsum('hdf,hg->hdgf', m, eye8).reshape(128, 128)

    for l in range(_LAYERS):
        wke, bke, bdrm = {}, {}, {}
        for e, (s, d, E) in enumerate(_EM):
            bd_a = blockdiag(rel_att[l, e] * (rel_pri[l, e] / 4.0)[:, None, None])
            wke[e] = lin_w[l, s, 0] @ bd_a
            bke[e] = lin_b[l, s, 0] @ bd_a
            bdrm[e] = blockdiag(rel_msg[l, e])

        Q, KV = {}, {}
        for t in range(4):
            ws = [lin_w[l, t, 1]]
            bs = [lin_b[l, t, 1]]
            widths = [128]
            for e in _SRC_EDGES[t]:
                ws += [wke[e], lin_w[l, t, 2]]
                bs += [bke[e], lin_b[l, t, 2]]
                widths.append(256)
            wcat = jnp.concatenate(ws, axis=1)
            bcat = jnp.zeros((8, wcat.shape[1]), jnp.float32).at[0].set(
                jnp.concatenate(bs, axis=0))
            outs = _stage1(xs[t], wcat, bcat, tuple(widths))
            Q[t] = outs[0]
            for j, e in enumerate(_SRC_EDGES[t]):
                KV[e] = outs[1 + j]

        plan = tuple((E, epads[e], nchunks[e]) for e, (s, d, E) in enumerate(_EM))
        dst_types = tuple(d for (s, d, E) in _EM)
        MSG = _make_pass_a(plan, dst_types)(
            *[KV[e] for e in range(8)], *[Q[t] for t in range(4)],
            *srcs, *dsts)
        planb = tuple((epads[e], epads[e] // (16 * _CH),
                       -(-_N_NODES[d] // _SWEEP))
                      for e, (s, d, E) in enumerate(_EM))
        AGG = _make_pass_b(planb)(*MSG, *dsts, zero_init)

        new_xs = []
        for t in range(4):
            a = jax.nn.sigmoid(skip[l, t]).reshape(1, 1).astype(jnp.float32)
            b3 = jnp.zeros((8, 128), jnp.float32).at[0].set(lin_b[l, t, 3])
            aggs = [AGG[e] for e in _IN_EDGES[t]]
            bds = [bdrm[e] for e in _IN_EDGES[t]]
            new_xs.append(_stage3(xs[t], a, rmat, lin_w[l, t, 3], b3, aggs, bds))
        xs = new_xs

    return tuple(xs)


# speedup vs baseline: 19.4425x; 1.0057x over previous
import functools

import jax
import jax.numpy as jnp
from jax import lax
from jax.experimental import pallas as pl
from jax.experimental.pallas import tpu as pltpu
from jax.experimental.pallas import tpu_sc as plsc

_N_NODES = [10000, 100000, 50000, 10000]
_EM = [(0, 1, 100000), (1, 1, 100000), (1, 2, 50000), (2, 3, 20000),
       (2, 2, 50000), (1, 0, 100000), (2, 1, 50000), (3, 2, 20000)]
_HID, _HEADS, _DH, _LAYERS = 128, 8, 16, 2
_CH = 128
_ROWS_SC = 10080
_ROWS_TILE = _ROWS_SC // 16
_SWEEP = 2 * _ROWS_SC

_SRC_EDGES = {0: [0], 1: [1, 2, 5], 2: [3, 4, 6], 3: [7]}
_IN_EDGES = {0: [5], 1: [0, 1, 6], 2: [2, 4, 7], 3: [3]}


def _epad(E):
    n_chunks = -(-E // (32 * _CH))
    return 32 * _CH * n_chunks, n_chunks


def _stage1(x, wcat, bcat, widths):
    N = x.shape[0]
    BN = 400
    offs = [0]
    for w in widths:
        offs.append(offs[-1] + w)
    W = offs[-1]

    def body(x_ref, w_ref, b_ref, *o_refs):
        xv = x_ref[...]
        for k in range(len(widths)):
            o_refs[k][...] = (
                jnp.dot(xv, w_ref[:, offs[k]:offs[k + 1]],
                        preferred_element_type=jnp.float32)
                + b_ref[0:1, offs[k]:offs[k + 1]])

    return pl.pallas_call(
        body,
        grid=(N // BN,),
        in_specs=[
            pl.BlockSpec((BN, 128), lambda i: (i, 0)),
            pl.BlockSpec((128, W), lambda i: (0, 0)),
            pl.BlockSpec((8, W), lambda i: (0, 0)),
        ],
        out_specs=[pl.BlockSpec((BN, w), lambda i: (i, 0)) for w in widths],
        out_shape=[jax.ShapeDtypeStruct((N, w), jnp.float32) for w in widths],
    )(x, wcat, bcat)


@functools.lru_cache(maxsize=None)
def _make_pass_a(plan, dst_types):
    def body(*refs):
        kvs = refs[0:8]
        qs = refs[8:12]
        srcs = refs[12:20]
        dsts = refs[20:28]
        msgs = refs[28:36]
        (srcv0, dstv0, kvrows0, qrows0, srcv1, dstv1, kvrows1, qrows1,
         atb, msgb) = refs[36:46]
        sems = refs[46:54]
        cid = lax.axis_index("c")
        sid = lax.axis_index("s")
        wid = sid * 2 + cid
        iota16 = lax.iota(jnp.int32, 16)
        bufs = [(srcv0, dstv0, kvrows0, qrows0, sems[0:4]),
                (srcv1, dstv1, kvrows1, qrows1, sems[4:8])]

        for e in range(8):
            _pa_section(kvs[e], qs[dst_types[e]], srcs[e], dsts[e], msgs[e],
                        plan[e], wid, iota16, bufs, atb, msgb)

    def _pa_section(kv_hbm, q_hbm, src_hbm, dst_hbm, msg_hbm,
                    eplan, wid, iota16, bufs, atb, msgb):
        E, E_pad, n_chunks = eplan

        def cbase(ci):
            return (wid * n_chunks + ci) * _CH

        def issue_idx(ci, p):
            srcv, dstv, _, _, (si, di, _, _) = bufs[p]
            pltpu.async_copy(src_hbm.at[pl.ds(cbase(ci), _CH)], srcv.at[0], si)
            pltpu.async_copy(dst_hbm.at[pl.ds(cbase(ci), _CH)], dstv.at[0], di)

        def issue_gather(ci, p):
            srcv, dstv, kvrows, qrows, (si, di, gk, gq) = bufs[p]
            pltpu.make_async_copy(src_hbm.at[pl.ds(cbase(ci), _CH)],
                                  srcv.at[0], si).wait()
            pltpu.make_async_copy(dst_hbm.at[pl.ds(cbase(ci), _CH)],
                                  dstv.at[0], di).wait()
            pltpu.async_copy(kv_hbm.at[srcv.at[0]], kvrows, gk)
            pltpu.async_copy(q_hbm.at[dstv.at[0]], qrows, gq)

        def compute(ci, p):
            srcv, dstv, kvrows, qrows, (si, di, gk, gq) = bufs[p]
            base = cbase(ci)
            pltpu.make_async_copy(kv_hbm.at[srcv.at[0]], kvrows, gk).wait()
            pltpu.make_async_copy(q_hbm.at[dstv.at[0]], qrows, gq).wait()

            def group_body(g, c2):
                rid = g * 16 + iota16
                valid = (base + rid) < E

                def head_body(h, c3):
                    acc = jnp.zeros((16,), jnp.float32)
                    for j in range(16):
                        col = h * 16 + ((iota16 + j) & 15)
                        acc = acc + (plsc.load_gather(qrows, [rid, col])
                                     * plsc.load_gather(kvrows, [rid, col]))
                    ex = jnp.where(valid, jnp.exp(acc), 0.0)
                    plsc.store_scatter(atb, [rid, jnp.full((16,), h, jnp.int32)], ex)
                    return c3

                lax.fori_loop(0, 8, head_body, 0)
                return c2

            lax.fori_loop(0, _CH // 16, group_body, 0)

            def row_body(r, c2):
                rv = jnp.full((16,), r, jnp.int32)
                av = plsc.load_gather(atb, [rv, iota16])
                av = jnp.where(iota16 < 8, av, 0.0)
                plsc.store_scatter(msgb, [rv, 128 + iota16], av)
                dnums = lax.GatherDimensionNumbers(
                    offset_dims=(), collapsed_slice_dims=(0,),
                    start_index_map=(0,))

                def hrow_body(h, c3):
                    sc = lax.gather(av, jnp.full((16, 1), h, jnp.int32), dnums,
                                    (1,), mode=lax.GatherScatterMode.PROMISE_IN_BOUNDS)
                    vv = plsc.load_gather(kvrows, [rv, 128 + h * 16 + iota16])
                    plsc.store_scatter(msgb, [rv, h * 16 + iota16], vv * sc)
                    return c3

                lax.fori_loop(0, 8, hrow_body, 0)
                return c2

            lax.fori_loop(0, _CH, row_body, 0)
            pltpu.sync_copy(msgb, msg_hbm.at[pl.ds(base, _CH)])

        issue_idx(0, 0)
        issue_gather(0, 0)

        def step(k, carry):
            c0 = 2 * k
            c1 = 2 * k + 1

            @pl.when(c1 < n_chunks)
            def _():
                issue_idx(c1, 1)
                issue_gather(c1, 1)
            compute(c0, 0)

            @pl.when(c1 < n_chunks)
            def _():
                @pl.when(c1 + 1 < n_chunks)
                def _():
                    issue_idx(c1 + 1, 0)
                    issue_gather(c1 + 1, 0)
                compute(c1, 1)
            return carry

        lax.fori_loop(0, (n_chunks + 1) // 2, step, 0)

    return pl.kernel(
        body,
        out_type=[jax.ShapeDtypeStruct((ep[1], 144), jnp.float32) for ep in plan],
        mesh=plsc.VectorSubcoreMesh(core_axis_name="c", subcore_axis_name="s"),
        compiler_params=pltpu.CompilerParams(needs_layout_passes=False,
                                             use_tc_tiling_on_sc=False),
        scratch_types=[
            pltpu.VMEM((1, _CH), jnp.int32),
            pltpu.VMEM((1, _CH), jnp.int32),
            pltpu.VMEM((_CH, 256), jnp.float32),
            pltpu.VMEM((_CH, 128), jnp.float32),
            pltpu.VMEM((1, _CH), jnp.int32),
            pltpu.VMEM((1, _CH), jnp.int32),
            pltpu.VMEM((_CH, 256), jnp.float32),
            pltpu.VMEM((_CH, 128), jnp.float32),
            pltpu.VMEM((_CH, 16), jnp.float32),
            pltpu.VMEM((_CH, 144), jnp.float32),
        ] + [pltpu.SemaphoreType.DMA] * 8,
    )


@functools.lru_cache(maxsize=None)
def _make_pass_b(planb):
    STG = 272

    def body(*refs):
        msgs = refs[0:8]
        dsts = refs[8:16]
        zero_hbm = refs[16]
        aggs = refs[17:25]
        acc_sh, dstv0, dstv1, eid_st, ldst_st, eidb, ldstb, wbuf = refs[25:33]
        sem1, dv0, dv1 = refs[33:36]
        cid = lax.axis_index("c")
        sid = lax.axis_index("s")
        iota16 = lax.iota(jnp.int32, 16)
        zi = jnp.zeros((16,), jnp.int32)
        junk = jnp.full((16,), _ROWS_SC, jnp.int32)
        dbufs = [(dstv0, dv0), (dstv1, dv1)]
        for e in range(8):
            _pb_section(msgs[e], dsts[e], zero_hbm, aggs[e], planb[e],
                        cid, sid, iota16, zi, junk, dbufs,
                        acc_sh, eid_st, ldst_st, eidb, ldstb, wbuf, sem1)

    def _pb_section(msg_hbm, dst_hbm, zero_hbm, agg_hbm, eplan,
                    cid, sid, iota16, zi, junk, dbufs,
                    acc_sh, eid_st, ldst_st, eidb, ldstb, wbuf, sem1):
        E_pad, n_chunks_sc, n_sweeps = eplan

        def init_body(t, c):
            eid_st[pl.ds(t * 16, 16)] = zi
            ldst_st[pl.ds(t * 16, 16)] = junk
            return c

        lax.fori_loop(0, STG // 16, init_body, 0)

        def fire_batch():
            def cp_body(t, c):
                sl = pl.ds(t * 16, 16)
                eidb[0, sl] = eid_st[sl]
                ldstb[0, sl] = ldst_st[sl]
                return c

            lax.fori_loop(0, 8, cp_body, 0)
            pltpu.async_copy(msg_hbm.at[eidb.at[0]], wbuf, sem1).wait()
            pltpu.sync_copy(wbuf, acc_sh.at[ldstb.at[0]], add=True)

        def cbase(ci):
            return (sid * n_chunks_sc + ci) * _CH

        def issue_dst(ci, p):
            dstv, dv = dbufs[p]
            pltpu.async_copy(dst_hbm.at[pl.ds(cbase(ci), _CH)], dstv.at[0], dv)

        for s in range(n_sweeps):
            lo = (s * 2 + cid) * _ROWS_SC
            pltpu.sync_copy(zero_hbm, acc_sh.at[pl.ds(sid * _ROWS_TILE, _ROWS_TILE)])
            plsc.subcore_barrier()
            issue_dst(0, 0)

            def half_chunk(ci, p, F, lo=lo):
                dstv, dv = dbufs[p]
                base = cbase(ci)
                pltpu.make_async_copy(dst_hbm.at[pl.ds(base, _CH)],
                                      dstv.at[0], dv).wait()

                @pl.when(ci + 1 < n_chunks_sc)
                def _():
                    issue_dst(ci + 1, 1 - p)

                def group_body(g, off):
                    rid = g * 16 + iota16
                    dvec = dstv[0, pl.ds(g * 16, 16)]
                    loc = dvec - lo
                    m = (loc >= 0) & (loc < _ROWS_SC)
                    plsc.store_compressed(eid_st.at[pl.ds(off, 16)],
                                          base + rid, mask=m)
                    plsc.store_compressed(ldst_st.at[pl.ds(off, 16)],
                                          loc, mask=m)
                    return off + jnp.sum(m.astype(jnp.int32))

                F = lax.fori_loop(0, _CH // 16, group_body, F)

                def with_batch(F):
                    fire_batch()

                    def shift_body(t, c):
                        dst_sl = pl.ds(t * 16, 16)
                        src_sl = pl.ds(128 + t * 16, 16)
                        eid_st[dst_sl] = eid_st[src_sl]
                        ldst_st[dst_sl] = ldst_st[src_sl]
                        return c

                    lax.fori_loop(0, 8, shift_body, 0)
                    return F - 128

                return lax.cond(F >= 128, with_batch, lambda F: F, F)

            def chunk_pair(k, F):
                F = half_chunk(2 * k, 0, F)
                return half_chunk(2 * k + 1, 1, F)

            F = lax.fori_loop(0, n_chunks_sc // 2, chunk_pair, jnp.int32(0))

            def flush_body(t, c):
                pos = t * 16 + iota16
                sl = pl.ds(t * 16, 16)
                ldst_st[sl] = jnp.where(pos < F, ldst_st[sl], junk)
                return c

            lax.fori_loop(0, 8, flush_body, 0)
            fire_batch()
            plsc.subcore_barrier()
            gbase = s * _SWEEP + cid * _ROWS_SC + sid * _ROWS_TILE
            pltpu.sync_copy(acc_sh.at[pl.ds(sid * _ROWS_TILE, _ROWS_TILE)],
                            agg_hbm.at[pl.ds(gbase, _ROWS_TILE)])
            plsc.subcore_barrier()

    return pl.kernel(
        body,
        out_type=[jax.ShapeDtypeStruct((ep[2] * _SWEEP, 144), jnp.float32)
                  for ep in planb],
        mesh=plsc.VectorSubcoreMesh(core_axis_name="c", subcore_axis_name="s"),
        compiler_params=pltpu.CompilerParams(needs_layout_passes=False,
                                             use_tc_tiling_on_sc=False),
        scratch_types=[
            pltpu.VMEM_SHARED((_ROWS_SC + 8, 144), jnp.float32),
            pltpu.VMEM((1, _CH), jnp.int32),
            pltpu.VMEM((1, _CH), jnp.int32),
            pltpu.VMEM((STG,), jnp.int32),
            pltpu.VMEM((STG,), jnp.int32),
            pltpu.VMEM((1, _CH), jnp.int32),
            pltpu.VMEM((1, _CH), jnp.int32),
            pltpu.VMEM((_CH, 144), jnp.float32),
            pltpu.SemaphoreType.DMA,
            pltpu.SemaphoreType.DMA,
            pltpu.SemaphoreType.DMA,
        ],
    )


def _stage3(x, a, rmat, w3, b3, aggs, bds):
    ne = len(aggs)
    N = x.shape[0]
    BN = 400

    def body(x_ref, a_ref, r_ref, w3_ref, b3_ref, *rest):
        agg_refs = rest[:ne]
        bd_refs = rest[ne:2 * ne]
        o_ref = rest[2 * ne]
        R = r_ref[...]
        acc = jnp.zeros((BN, 128), jnp.float32)
        for i in range(ne):
            blk = agg_refs[i][...]
            num = blk[:, :128]
            den = blk[:, 128:136] + 1e-16
            denrep = jnp.dot(den, R, preferred_element_type=jnp.float32)
            acc = acc + jnp.dot(num / denrep, bd_refs[i][...],
                                preferred_element_type=jnp.float32)
        o = (jnp.dot(jax.nn.gelu(acc), w3_ref[...],
                     preferred_element_type=jnp.float32) + b3_ref[0:1, :])
        av = a_ref[0, 0]
        y = av * o + (1.0 - av) * x_ref[...]
        o_ref[...] = jnp.maximum(y, 0.0)

    in_specs = [
        pl.BlockSpec((BN, 128), lambda i: (i, 0)),
        pl.BlockSpec(memory_space=pltpu.SMEM),
        pl.BlockSpec((8, 128), lambda i: (0, 0)),
        pl.BlockSpec((128, 128), lambda i: (0, 0)),
        pl.BlockSpec((8, 128), lambda i: (0, 0)),
    ]
    in_specs += [pl.BlockSpec((BN, 144), lambda i: (i, 0)) for _ in range(ne)]
    in_specs += [pl.BlockSpec((128, 128), lambda i: (0, 0)) for _ in range(ne)]
    return pl.pallas_call(
        body,
        grid=(N // BN,),
        in_specs=in_specs,
        out_specs=pl.BlockSpec((BN, 128), lambda i: (i, 0)),
        out_shape=jax.ShapeDtypeStruct((N, 128), jnp.float32),
    )(x, a, rmat, w3, b3, *aggs, *bds)


def kernel(x_document, x_word, x_medical_concept, x_symptom_category,
           ei_contains, ei_co_occurs, ei_maps_to, ei_belongs_to, ei_related_to,
           ei_rev_contains, ei_rev_maps_to, ei_rev_belongs_to,
           lin_w, lin_b, rel_att, rel_msg, rel_pri, skip):
    xs = [x_document, x_word, x_medical_concept, x_symptom_category]
    eis = [ei_contains, ei_co_occurs, ei_maps_to, ei_belongs_to, ei_related_to,
           ei_rev_contains, ei_rev_maps_to, ei_rev_belongs_to]

    srcs, dsts, epads, nchunks = [], [], [], []
    for e, (s, d, E) in enumerate(_EM):
        ep, nc = _epad(E)
        src = eis[e][0].astype(jnp.int32)
        dst = eis[e][1].astype(jnp.int32)
        srcs.append(jnp.pad(src, (0, ep - E)))
        dsts.append(jnp.pad(dst, (0, ep - E)))
        epads.append(ep)
        nchunks.append(nc)

    eye8 = jnp.eye(8, dtype=jnp.float32)
    rmat = jnp.repeat(eye8, 16, axis=1)
    zero_init = jnp.zeros((_ROWS_TILE, 144), jnp.float32)

    def blockdiag(m):
        return jnp.einsum('hdf,hg->hdgf', m, eye8).reshape(128, 128)

    for l in range(_LAYERS):
        wke, bke, bdrm = {}, {}, {}
        for e, (s, d, E) in enumerate(_EM):
            bd_a = blockdiag(rel_att[l, e] * (rel_pri[l, e] / 4.0)[:, None, None])
            wke[e] = lin_w[l, s, 0] @ bd_a
            bke[e] = lin_b[l, s, 0] @ bd_a
            bdrm[e] = blockdiag(rel_msg[l, e])

        Q, KV = {}, {}
        for t in range(4):
            ws = [lin_w[l, t, 1]]
            bs = [lin_b[l, t, 1]]
            widths = [128]
            for e in _SRC_EDGES[t]:
                ws += [wke[e], lin_w[l, t, 2]]
                bs += [bke[e], lin_b[l, t, 2]]
                widths.append(256)
            wcat = jnp.concatenate(ws, axis=1)
            bcat = jnp.zeros((8, wcat.shape[1]), jnp.float32).at[0].set(
                jnp.concatenate(bs, axis=0))
            outs = _stage1(xs[t], wcat, bcat, tuple(widths))
            Q[t] = outs[0]
            for j, e in enumerate(_SRC_EDGES[t]):
                KV[e] = outs[1 + j]

        plan = tuple((E, epads[e], nchunks[e]) for e, (s, d, E) in enumerate(_EM))
        dst_types = tuple(d for (s, d, E) in _EM)
        MSG = _make_pass_a(plan, dst_types)(
            *[KV[e] for e in range(8)], *[Q[t] for t in range(4)],
            *srcs, *dsts)
        planb = tuple((epads[e], epads[e] // (16 * _CH),
                       -(-_N_NODES[d] // _SWEEP))
                      for e, (s, d, E) in enumerate(_EM))
        AGG = _make_pass_b(planb)(*MSG, *dsts, zero_init)

        new_xs = []
        for t in range(4):
            a = jax.nn.sigmoid(skip[l, t]).reshape(1, 1).astype(jnp.float32)
            b3 = jnp.zeros((8, 128), jnp.float32).at[0].set(lin_b[l, t, 3])
            aggs = [AGG[e] for e in _IN_EDGES[t]]
            bds = [bdrm[e] for e in _IN_EDGES[t]]
            new_xs.append(_stage3(xs[t], a, rmat, lin_w[l, t, 3], b3, aggs, bds))
        xs = new_xs

    return tuple(xs)
